# Initial kernel scaffold; baseline (speedup 1.0000x reference)
import functools

import jax
import jax.numpy as jnp
from jax import lax
from jax.experimental import pallas as pl
from jax.experimental.pallas import tpu as pltpu
from jax.experimental.pallas import tpu_sc as plsc

H = 128
N = 10000
E = 320000
L = 16
CH = 128
NW = 32
NCH = E // CH
RB = 125
NB = 1000
BE = 512

_mesh = plsc.VectorSubcoreMesh(core_axis_name="c", subcore_axis_name="s")


def _silu(x):
    return x * jax.nn.sigmoid(x)


def _tc_pre(h, WiaT, WibT, be1):
    def body(h_r, wa_r, wb_r, b_r, A_r, B_r):
        hb = h_r[...]
        A_r[...] = jnp.dot(hb, wa_r[...], preferred_element_type=jnp.float32) + b_r[...]
        B_r[...] = jnp.dot(hb, wb_r[...], preferred_element_type=jnp.float32)

    return pl.pallas_call(
        body,
        grid=(N // NB,),
        in_specs=[
            pl.BlockSpec((NB, H), lambda i: (i, 0)),
            pl.BlockSpec((H, H), lambda i: (0, 0)),
            pl.BlockSpec((H, H), lambda i: (0, 0)),
            pl.BlockSpec((1, H), lambda i: (0, 0)),
        ],
        out_specs=[
            pl.BlockSpec((NB, H), lambda i: (i, 0)),
            pl.BlockSpec((NB, H), lambda i: (i, 0)),
        ],
        out_shape=[jax.ShapeDtypeStruct((N, H), jnp.float32)] * 2,
    )(h, WiaT, WibT, be1)


def _sc_gather(A, B, pos, src, dst):
    @functools.partial(
        pl.kernel,
        out_type=[
            jax.ShapeDtypeStruct((E, H), jnp.float32),
            jax.ShapeDtypeStruct((E,), jnp.float32),
            jax.ShapeDtypeStruct((E,), jnp.float32),
            jax.ShapeDtypeStruct((E,), jnp.float32),
            jax.ShapeDtypeStruct((E,), jnp.float32),
        ],
        mesh=_mesh,
        scratch_types=[
            pltpu.VMEM((N, 3), jnp.float32),
            pltpu.VMEM((CH,), jnp.int32),
            pltpu.VMEM((CH,), jnp.int32),
            pltpu.VMEM((CH, H), jnp.float32),
            pltpu.VMEM((CH, H), jnp.float32),
            pltpu.VMEM((CH,), jnp.float32),
            pltpu.VMEM((CH,), jnp.float32),
            pltpu.VMEM((CH,), jnp.float32),
            pltpu.VMEM((CH,), jnp.float32),
            pltpu.SemaphoreType.DMA,
            pltpu.SemaphoreType.DMA,
        ],
    )
    def k(A_h, B_h, pos_h, src_h, dst_h, G_h, r2_h, dx_h, dy_h, dz_h,
          posv, dbuf, sbuf, bufA, bufB, r2b, dxb, dyb, dzb, semA, semB):
        cid = lax.axis_index("c")
        sid = lax.axis_index("s")
        wid = sid * 2 + cid
        nk = 78 + jnp.where(wid < NCH - 78 * NW, 1, 0)
        pltpu.sync_copy(pos_h, posv)

        def body(kk, carry):
            g = wid + kk * NW
            base = g * CH
            pltpu.sync_copy(dst_h.at[pl.ds(base, CH)], dbuf)
            pltpu.sync_copy(src_h.at[pl.ds(base, CH)], sbuf)
            cpA = pltpu.async_copy(A_h.at[dbuf], bufA, semA)
            cpB = pltpu.async_copy(B_h.at[sbuf], bufB, semB)
            for gi in range(CH // L):
                sl = pl.ds(gi * L, L)
                di = dbuf[sl]
                si = sbuf[sl]
                c0 = jnp.zeros((L,), jnp.int32)
                c1 = jnp.full((L,), 1, jnp.int32)
                c2 = jnp.full((L,), 2, jnp.int32)
                ddx = plsc.load_gather(posv, [di, c0]) - plsc.load_gather(posv, [si, c0])
                ddy = plsc.load_gather(posv, [di, c1]) - plsc.load_gather(posv, [si, c1])
                ddz = plsc.load_gather(posv, [di, c2]) - plsc.load_gather(posv, [si, c2])
                dxb[sl] = ddx
                dyb[sl] = ddy
                dzb[sl] = ddz
                r2b[sl] = ddx * ddx + ddy * ddy + ddz * ddz
            cpA.wait()
            cpB.wait()

            def row(r, c):
                for q in range(H // L):
                    s2 = pl.ds(q * L, L)
                    bufA[r, s2] = bufA[r, s2] + bufB[r, s2]
                return c

            lax.fori_loop(0, CH, row, 0)
            pltpu.sync_copy(bufA, G_h.at[pl.ds(base, CH)])
            pltpu.sync_copy(r2b, r2_h.at[pl.ds(base, CH)])
            pltpu.sync_copy(dxb, dx_h.at[pl.ds(base, CH)])
            pltpu.sync_copy(dyb, dy_h.at[pl.ds(base, CH)])
            pltpu.sync_copy(dzb, dz_h.at[pl.ds(base, CH)])
            return carry

        lax.fori_loop(0, nk, body, 0)

    return k(A, B, pos, src, dst)


def _tc_edge(G, ea, r2c, WeET, wr_row, We2T, be2, Wx1T, bx1, wx2_row, bx2):
    def body(G_r, ea_r, r2_r, weet_r, wr_r, we2t_r, be2_r, wx1t_r, bx1_r,
             wx2_r, bx2_r, m_r, gs_r):
        r2 = r2_r[...]
        pre1 = (G_r[...] + r2 * wr_r[...]
                + jnp.dot(ea_r[...], weet_r[...], preferred_element_type=jnp.float32))
        l1 = _silu(pre1)
        mm = _silu(jnp.dot(l1, we2t_r[...], preferred_element_type=jnp.float32) + be2_r[...])
        m_r[...] = mm
        g1 = _silu(jnp.dot(mm, wx1t_r[...], preferred_element_type=jnp.float32) + bx1_r[...])
        gate = jnp.tanh(jnp.sum(g1 * wx2_r[...], axis=1, keepdims=True) + bx2_r[...])
        gs_r[...] = gate / (r2 + 1.0)

    return pl.pallas_call(
        body,
        grid=(E // BE,),
        in_specs=[
            pl.BlockSpec((BE, H), lambda i: (i, 0)),
            pl.BlockSpec((BE, 16), lambda i: (i, 0)),
            pl.BlockSpec((BE, 1), lambda i: (i, 0)),
            pl.BlockSpec((16, H), lambda i: (0, 0)),
            pl.BlockSpec((1, H), lambda i: (0, 0)),
            pl.BlockSpec((H, H), lambda i: (0, 0)),
            pl.BlockSpec((1, H), lambda i: (0, 0)),
            pl.BlockSpec((H, H), lambda i: (0, 0)),
            pl.BlockSpec((1, H), lambda i: (0, 0)),
            pl.BlockSpec((1, H), lambda i: (0, 0)),
            pl.BlockSpec((1, 1), lambda i: (0, 0)),
        ],
        out_specs=[
            pl.BlockSpec((BE, H), lambda i: (i, 0)),
            pl.BlockSpec((BE, 1), lambda i: (i, 0)),
        ],
        out_shape=[
            jax.ShapeDtypeStruct((E, H), jnp.float32),
            jax.ShapeDtypeStruct((E, 1), jnp.float32),
        ],
    )(G, ea, r2c, WeET, wr_row, We2T, be2, Wx1T, bx1, wx2_row, bx2)


def _sc_scatter(m, gs, dx, dy, dz, dst):
    @functools.partial(
        pl.kernel,
        out_type=[
            jax.ShapeDtypeStruct((2, N, H), jnp.float32),
            jax.ShapeDtypeStruct((2, N, L), jnp.float32),
        ],
        mesh=_mesh,
        scratch_types=[
            pltpu.VMEM_SHARED((N, H), jnp.float32),
            pltpu.VMEM_SHARED((N, L), jnp.float32),
            pltpu.VMEM((CH,), jnp.int32),
            pltpu.VMEM((CH, H), jnp.float32),
            pltpu.VMEM((CH, L), jnp.float32),
            pltpu.VMEM((CH,), jnp.float32),
            pltpu.VMEM((CH,), jnp.float32),
            pltpu.VMEM((CH,), jnp.float32),
            pltpu.VMEM((CH,), jnp.float32),
            pltpu.VMEM((RB, H), jnp.float32),
            pltpu.VMEM((RB, L), jnp.float32),
        ],
    )
    def k(m_h, gs_h, dx_h, dy_h, dz_h, dst_h, agg_h, dp_h,
          agg_s, dp_s, dbuf, mbuf, cbuf, gsbuf, dxb, dyb, dzb, zb, zb16):
        cid = lax.axis_index("c")
        sid = lax.axis_index("s")
        wid = sid * 2 + cid
        zv = jnp.zeros((L,), jnp.float32)

        def zrow(r, c):
            for q in range(H // L):
                zb[r, pl.ds(q * L, L)] = zv
            zb16[r, pl.ds(0, L)] = zv
            return c

        lax.fori_loop(0, RB, zrow, 0)

        def crow(r, c):
            cbuf[r, pl.ds(0, L)] = zv
            return c

        lax.fori_loop(0, CH, crow, 0)
        for j in range(5):
            r0 = sid * 625 + j * RB
            pltpu.sync_copy(zb, agg_s.at[pl.ds(r0, RB)])
            pltpu.sync_copy(zb16, dp_s.at[pl.ds(r0, RB)])
        plsc.subcore_barrier()
        nk = 78 + jnp.where(wid < NCH - 78 * NW, 1, 0)

        def body(kk, c):
            g = wid + kk * NW
            base = g * CH
            pltpu.sync_copy(dst_h.at[pl.ds(base, CH)], dbuf)
            pltpu.sync_copy(m_h.at[pl.ds(base, CH)], mbuf)
            pltpu.sync_copy(gs_h.at[pl.ds(base, CH)], gsbuf)
            pltpu.sync_copy(dx_h.at[pl.ds(base, CH)], dxb)
            pltpu.sync_copy(dy_h.at[pl.ds(base, CH)], dyb)
            pltpu.sync_copy(dz_h.at[pl.ds(base, CH)], dzb)
            c0 = jnp.zeros((L,), jnp.int32)
            c1 = jnp.full((L,), 1, jnp.int32)
            c2 = jnp.full((L,), 2, jnp.int32)
            for gi in range(CH // L):
                sl = pl.ds(gi * L, L)
                gsv = gsbuf[sl]
                rows = lax.iota(jnp.int32, L) + gi * L
                plsc.store_scatter(cbuf, [rows, c0], dxb[sl] * gsv)
                plsc.store_scatter(cbuf, [rows, c1], dyb[sl] * gsv)
                plsc.store_scatter(cbuf, [rows, c2], dzb[sl] * gsv)
            pltpu.sync_copy(mbuf, agg_s.at[dbuf], add=True)
            pltpu.sync_copy(cbuf, dp_s.at[dbuf], add=True)
            return c

        lax.fori_loop(0, nk, body, 0)
        plsc.subcore_barrier()
        for j in range(5):
            r0 = sid * 625 + j * RB
            pltpu.sync_copy(agg_s.at[pl.ds(r0, RB)], zb)
            pltpu.sync_copy(zb, agg_h.at[cid, pl.ds(r0, RB)])
            pltpu.sync_copy(dp_s.at[pl.ds(r0, RB)], zb16)
            pltpu.sync_copy(zb16, dp_h.at[cid, pl.ds(r0, RB)])

    return k(m, gs, dx, dy, dz, dst)


def _tc_node(h, agg2, dp2, pos, Wh1aT, Wh1bT, bh1, Wh2T, bh2, g, b):
    def body(h_r, agg_r, dp_r, pos_r, wa_r, wb_r, b1_r, w2_r, b2_r, g_r, be_r,
             ho_r, po_r):
        hb = h_r[...]
        a3 = agg_r[...]
        agg = a3[0] + a3[1]
        t = _silu(jnp.dot(hb, wa_r[...], preferred_element_type=jnp.float32)
                  + jnp.dot(agg, wb_r[...], preferred_element_type=jnp.float32)
                  + b1_r[...])
        dh = jnp.dot(t, w2_r[...], preferred_element_type=jnp.float32) + b2_r[...]
        x = hb + dh
        mu = jnp.mean(x, axis=1, keepdims=True)
        var = jnp.mean((x - mu) ** 2, axis=1, keepdims=True)
        ho_r[...] = (x - mu) * lax.rsqrt(var + 1e-5) * g_r[...] + be_r[...]
        d3 = dp_r[...]
        dp = d3[0] + d3[1]
        po_r[...] = pos_r[...] + dp[:, 0:3]

    return pl.pallas_call(
        body,
        grid=(N // NB,),
        in_specs=[
            pl.BlockSpec((NB, H), lambda i: (i, 0)),
            pl.BlockSpec((2, NB, H), lambda i: (0, i, 0)),
            pl.BlockSpec((2, NB, L), lambda i: (0, i, 0)),
            pl.BlockSpec((NB, 3), lambda i: (i, 0)),
            pl.BlockSpec((H, H), lambda i: (0, 0)),
            pl.BlockSpec((H, H), lambda i: (0, 0)),
            pl.BlockSpec((1, H), lambda i: (0, 0)),
            pl.BlockSpec((H, H), lambda i: (0, 0)),
            pl.BlockSpec((1, H), lambda i: (0, 0)),
            pl.BlockSpec((1, H), lambda i: (0, 0)),
            pl.BlockSpec((1, H), lambda i: (0, 0)),
        ],
        out_specs=[
            pl.BlockSpec((NB, H), lambda i: (i, 0)),
            pl.BlockSpec((NB, 3), lambda i: (i, 0)),
        ],
        out_shape=[
            jax.ShapeDtypeStruct((N, H), jnp.float32),
            jax.ShapeDtypeStruct((N, 3), jnp.float32),
        ],
    )(h, agg2, dp2, pos, Wh1aT, Wh1bT, bh1, Wh2T, bh2, g, b)


def kernel(h, pos, edge_index, edge_attr, We1_w, We1_b, We2_w, We2_b,
           Wh1_w, Wh1_b, Wh2_w, Wh2_b, Wx1_w, Wx1_b, Wx2_w, Wx2_b, ln_g, ln_b):
    src = edge_index[0].astype(jnp.int32)
    dst = edge_index[1].astype(jnp.int32)
    WiaT = We1_w[:, :H].T
    WibT = We1_w[:, H:2 * H].T
    wr_row = We1_w[:, 2 * H:2 * H + 1].T
    WeET = We1_w[:, 2 * H + 1:].T
    A, Bm = _tc_pre(h, WiaT, WibT, We1_b.reshape(1, H))
    G, r2v, dxv, dyv, dzv = _sc_gather(A, Bm, pos, src, dst)
    m, gs = _tc_edge(G, edge_attr, r2v.reshape(E, 1), WeET, wr_row,
                     We2_w.T, We2_b.reshape(1, H), Wx1_w.T, Wx1_b.reshape(1, H),
                     Wx2_w, Wx2_b.reshape(1, 1))
    agg2, dp2 = _sc_scatter(m, gs.reshape(E), dxv, dyv, dzv, dst)
    h_out, pos_out = _tc_node(h, agg2, dp2, pos, Wh1_w[:, :H].T, Wh1_w[:, H:].T,
                              Wh1_b.reshape(1, H), Wh2_w.T, Wh2_b.reshape(1, H),
                              ln_g.reshape(1, H), ln_b.reshape(1, H))
    return (h_out, pos_out)


# speedup vs baseline: 2.9778x; 2.9778x over previous
import functools

import jax
import jax.numpy as jnp
from jax import lax
from jax.experimental import pallas as pl
from jax.experimental.pallas import tpu as pltpu
from jax.experimental.pallas import tpu_sc as plsc

H = 128
N = 10000
E = 320000
L = 16
CH = 128
NW = 32
NCH = E // CH
NP = 10240
RB = 128
NB = 1000
BE = 512

_mesh = plsc.VectorSubcoreMesh(core_axis_name="c", subcore_axis_name="s")


def _silu(x):
    return x * jax.nn.sigmoid(x)


def _tc_pre(h, WiaT, WibT, be1):
    def body(h_r, wa_r, wb_r, b_r, A_r, B_r):
        hb = h_r[...]
        A_r[...] = jnp.dot(hb, wa_r[...], preferred_element_type=jnp.float32) + b_r[...]
        B_r[...] = jnp.dot(hb, wb_r[...], preferred_element_type=jnp.float32)

    return pl.pallas_call(
        body,
        grid=(N // NB,),
        in_specs=[
            pl.BlockSpec((NB, H), lambda i: (i, 0)),
            pl.BlockSpec((H, H), lambda i: (0, 0)),
            pl.BlockSpec((H, H), lambda i: (0, 0)),
            pl.BlockSpec((1, H), lambda i: (0, 0)),
        ],
        out_specs=[
            pl.BlockSpec((NB, H), lambda i: (i, 0)),
            pl.BlockSpec((NB, H), lambda i: (i, 0)),
        ],
        out_shape=[jax.ShapeDtypeStruct((N, H), jnp.float32)] * 2,
    )(h, WiaT, WibT, be1)


def _sc_gather(A, B, pos, src, dst):
    @functools.partial(
        pl.kernel,
        out_type=[
            jax.ShapeDtypeStruct((E, H), jnp.float32),
            jax.ShapeDtypeStruct((E,), jnp.float32),
            jax.ShapeDtypeStruct((E,), jnp.float32),
            jax.ShapeDtypeStruct((E,), jnp.float32),
            jax.ShapeDtypeStruct((E,), jnp.float32),
        ],
        mesh=_mesh,
        compiler_params=pltpu.CompilerParams(needs_layout_passes=False),
        scratch_types=[
            pltpu.VMEM((3 * N,), jnp.float32),
            pltpu.VMEM((CH,), jnp.int32),
            pltpu.VMEM((CH,), jnp.int32),
            pltpu.VMEM((CH, H), jnp.float32),
            pltpu.VMEM((CH, H), jnp.float32),
            pltpu.VMEM((CH,), jnp.float32),
            pltpu.VMEM((CH,), jnp.float32),
            pltpu.VMEM((CH,), jnp.float32),
            pltpu.VMEM((CH,), jnp.float32),
            pltpu.SemaphoreType.DMA,
            pltpu.SemaphoreType.DMA,
        ],
    )
    def k(A_h, B_h, pos_h, src_h, dst_h, G_h, r2_h, dx_h, dy_h, dz_h,
          posv, dbuf, sbuf, bufA, bufB, r2b, dxb, dyb, dzb, semA, semB):
        cid = lax.axis_index("c")
        sid = lax.axis_index("s")
        wid = sid * 2 + cid
        nk = 78 + jnp.where(wid < NCH - 78 * NW, 1, 0)
        pltpu.sync_copy(pos_h, posv)

        def body(kk, carry):
            g = wid + kk * NW
            base = g * CH
            pltpu.sync_copy(dst_h.at[pl.ds(base, CH)], dbuf)
            pltpu.sync_copy(src_h.at[pl.ds(base, CH)], sbuf)
            cpA = pltpu.async_copy(A_h.at[dbuf], bufA, semA)
            cpB = pltpu.async_copy(B_h.at[sbuf], bufB, semB)
            for gi in range(CH // L):
                sl = pl.ds(gi * L, L)
                di = dbuf[sl] * 3
                si = sbuf[sl] * 3
                ddx = plsc.load_gather(posv, [di]) - plsc.load_gather(posv, [si])
                ddy = plsc.load_gather(posv, [di + 1]) - plsc.load_gather(posv, [si + 1])
                ddz = plsc.load_gather(posv, [di + 2]) - plsc.load_gather(posv, [si + 2])
                dxb[sl] = ddx
                dyb[sl] = ddy
                dzb[sl] = ddz
                r2b[sl] = ddx * ddx + ddy * ddy + ddz * ddz
            cpA.wait()
            cpB.wait()

            def row(r, c):
                for q in range(H // L):
                    s2 = pl.ds(q * L, L)
                    bufA[r, s2] = bufA[r, s2] + bufB[r, s2]
                return c

            lax.fori_loop(0, CH, row, 0)
            pltpu.sync_copy(bufA, G_h.at[pl.ds(base, CH)])
            pltpu.sync_copy(r2b, r2_h.at[pl.ds(base, CH)])
            pltpu.sync_copy(dxb, dx_h.at[pl.ds(base, CH)])
            pltpu.sync_copy(dyb, dy_h.at[pl.ds(base, CH)])
            pltpu.sync_copy(dzb, dz_h.at[pl.ds(base, CH)])
            return carry

        lax.fori_loop(0, nk, body, 0)

    return k(A, B, pos, src, dst)


def _tc_edge(G, ea, r2c, WeET, wr_row, We2T, be2, Wx1T, bx1, wx2_row, bx2):
    def body(G_r, ea_r, r2_r, weet_r, wr_r, we2t_r, be2_r, wx1t_r, bx1_r,
             wx2_r, bx2_r, m_r, gs_r):
        r2 = r2_r[...]
        pre1 = (G_r[...] + r2 * wr_r[...]
                + jnp.dot(ea_r[...], weet_r[...], preferred_element_type=jnp.float32))
        l1 = _silu(pre1)
        mm = _silu(jnp.dot(l1, we2t_r[...], preferred_element_type=jnp.float32) + be2_r[...])
        m_r[...] = mm
        g1 = _silu(jnp.dot(mm, wx1t_r[...], preferred_element_type=jnp.float32) + bx1_r[...])
        gate = jnp.tanh(jnp.sum(g1 * wx2_r[...], axis=1, keepdims=True) + bx2_r[...])
        gs_r[...] = gate / (r2 + 1.0)

    return pl.pallas_call(
        body,
        grid=(E // BE,),
        in_specs=[
            pl.BlockSpec((BE, H), lambda i: (i, 0)),
            pl.BlockSpec((BE, 16), lambda i: (i, 0)),
            pl.BlockSpec((BE, 1), lambda i: (i, 0)),
            pl.BlockSpec((16, H), lambda i: (0, 0)),
            pl.BlockSpec((1, H), lambda i: (0, 0)),
            pl.BlockSpec((H, H), lambda i: (0, 0)),
            pl.BlockSpec((1, H), lambda i: (0, 0)),
            pl.BlockSpec((H, H), lambda i: (0, 0)),
            pl.BlockSpec((1, H), lambda i: (0, 0)),
            pl.BlockSpec((1, H), lambda i: (0, 0)),
            pl.BlockSpec((1, 1), lambda i: (0, 0)),
        ],
        out_specs=[
            pl.BlockSpec((BE, H), lambda i: (i, 0)),
            pl.BlockSpec((BE, 1), lambda i: (i, 0)),
        ],
        out_shape=[
            jax.ShapeDtypeStruct((E, H), jnp.float32),
            jax.ShapeDtypeStruct((E, 1), jnp.float32),
        ],
    )(G, ea, r2c, WeET, wr_row, We2T, be2, Wx1T, bx1, wx2_row, bx2)


def _sc_scatter(m, gs, dx, dy, dz, dst):
    @functools.partial(
        pl.kernel,
        out_type=[
            jax.ShapeDtypeStruct((2, NP, H), jnp.float32),
            jax.ShapeDtypeStruct((2, 3 * NP), jnp.float32),
        ],
        mesh=_mesh,
        compiler_params=pltpu.CompilerParams(needs_layout_passes=False),
        scratch_types=[
            pltpu.VMEM_SHARED((NP, H), jnp.float32),
            pltpu.VMEM_SHARED((3 * NP,), jnp.float32),
            pltpu.VMEM((CH,), jnp.int32),
            pltpu.VMEM((CH, H), jnp.float32),
            pltpu.VMEM((CH,), jnp.float32),
            pltpu.VMEM((CH,), jnp.float32),
            pltpu.VMEM((CH,), jnp.float32),
            pltpu.VMEM((CH,), jnp.float32),
            pltpu.VMEM((CH,), jnp.float32),
            pltpu.VMEM((CH,), jnp.float32),
            pltpu.VMEM((CH,), jnp.float32),
            pltpu.VMEM((CH,), jnp.int32),
            pltpu.VMEM((CH,), jnp.int32),
            pltpu.VMEM((CH,), jnp.int32),
            pltpu.VMEM((3 * NP // 16,), jnp.float32),
        ],
    )
    def k(m_h, gs_h, dx_h, dy_h, dz_h, dst_h, agg_h, dp_h,
          agg_s, dp_s, dbuf, mbuf, gsbuf, dxb, dyb, dzb,
          cxb, cyb, czb, ixb, iyb, izb, dpb):
        cid = lax.axis_index("c")
        sid = lax.axis_index("s")
        wid = sid * 2 + cid
        zv = jnp.zeros((L,), jnp.float32)
        DPW = 3 * NP // 16

        def zrow(r, c):
            for q in range(H // L):
                mbuf[r, pl.ds(q * L, L)] = zv
            return c

        lax.fori_loop(0, CH, zrow, 0)

        def zdp(r, c):
            dpb[pl.ds(r * L, L)] = zv
            return c

        lax.fori_loop(0, DPW // L, zdp, 0)
        for j in range(5):
            pltpu.sync_copy(mbuf, agg_s.at[pl.ds(sid * 640 + j * RB, RB)])
        pltpu.sync_copy(dpb, dp_s.at[pl.ds(sid * DPW, DPW)])
        plsc.subcore_barrier()
        nk = 78 + jnp.where(wid < NCH - 78 * NW, 1, 0)

        def body(kk, c):
            g = wid + kk * NW
            base = g * CH
            pltpu.sync_copy(dst_h.at[pl.ds(base, CH)], dbuf)
            pltpu.sync_copy(m_h.at[pl.ds(base, CH)], mbuf)
            pltpu.sync_copy(gs_h.at[pl.ds(base, CH)], gsbuf)
            pltpu.sync_copy(dx_h.at[pl.ds(base, CH)], dxb)
            pltpu.sync_copy(dy_h.at[pl.ds(base, CH)], dyb)
            pltpu.sync_copy(dz_h.at[pl.ds(base, CH)], dzb)
            for gi in range(CH // L):
                sl = pl.ds(gi * L, L)
                gsv = gsbuf[sl]
                i3 = dbuf[sl] * 3
                ixb[sl] = i3
                iyb[sl] = i3 + 1
                izb[sl] = i3 + 2
                cxb[sl] = dxb[sl] * gsv
                cyb[sl] = dyb[sl] * gsv
                czb[sl] = dzb[sl] * gsv
            pltpu.sync_copy(mbuf, agg_s.at[dbuf], add=True)
            pltpu.sync_copy(cxb, dp_s.at[ixb], add=True)
            pltpu.sync_copy(cyb, dp_s.at[iyb], add=True)
            pltpu.sync_copy(czb, dp_s.at[izb], add=True)
            return c

        lax.fori_loop(0, nk, body, 0)
        plsc.subcore_barrier()
        for j in range(5):
            r0 = sid * 640 + j * RB
            pltpu.sync_copy(agg_s.at[pl.ds(r0, RB)], mbuf)
            pltpu.sync_copy(mbuf, agg_h.at[cid, pl.ds(r0, RB)])
        pltpu.sync_copy(dp_s.at[pl.ds(sid * DPW, DPW)], dpb)
        pltpu.sync_copy(dpb, dp_h.at[cid, pl.ds(sid * DPW, DPW)])

    return k(m, gs, dx, dy, dz, dst)


def _tc_node(h, agg2, dp2, pos, Wh1aT, Wh1bT, bh1, Wh2T, bh2, g, b):
    def body(h_r, agg_r, dp_r, pos_r, wa_r, wb_r, b1_r, w2_r, b2_r, g_r, be_r,
             ho_r, po_r):
        hb = h_r[...]
        a3 = agg_r[...]
        agg = a3[0] + a3[1]
        t = _silu(jnp.dot(hb, wa_r[...], preferred_element_type=jnp.float32)
                  + jnp.dot(agg, wb_r[...], preferred_element_type=jnp.float32)
                  + b1_r[...])
        dh = jnp.dot(t, w2_r[...], preferred_element_type=jnp.float32) + b2_r[...]
        x = hb + dh
        mu = jnp.mean(x, axis=1, keepdims=True)
        var = jnp.mean((x - mu) ** 2, axis=1, keepdims=True)
        ho_r[...] = (x - mu) * lax.rsqrt(var + 1e-5) * g_r[...] + be_r[...]
        d3 = dp_r[...]
        po_r[...] = pos_r[...] + d3[0] + d3[1]

    return pl.pallas_call(
        body,
        grid=(N // NB,),
        in_specs=[
            pl.BlockSpec((NB, H), lambda i: (i, 0)),
            pl.BlockSpec((2, NB, H), lambda i: (0, i, 0)),
            pl.BlockSpec((2, NB, 3), lambda i: (0, i, 0)),
            pl.BlockSpec((NB, 3), lambda i: (i, 0)),
            pl.BlockSpec((H, H), lambda i: (0, 0)),
            pl.BlockSpec((H, H), lambda i: (0, 0)),
            pl.BlockSpec((1, H), lambda i: (0, 0)),
            pl.BlockSpec((H, H), lambda i: (0, 0)),
            pl.BlockSpec((1, H), lambda i: (0, 0)),
            pl.BlockSpec((1, H), lambda i: (0, 0)),
            pl.BlockSpec((1, H), lambda i: (0, 0)),
        ],
        out_specs=[
            pl.BlockSpec((NB, H), lambda i: (i, 0)),
            pl.BlockSpec((NB, 3), lambda i: (i, 0)),
        ],
        out_shape=[
            jax.ShapeDtypeStruct((N, H), jnp.float32),
            jax.ShapeDtypeStruct((N, 3), jnp.float32),
        ],
    )(h, agg2, dp2, pos, Wh1aT, Wh1bT, bh1, Wh2T, bh2, g, b)


def kernel(h, pos, edge_index, edge_attr, We1_w, We1_b, We2_w, We2_b,
           Wh1_w, Wh1_b, Wh2_w, Wh2_b, Wx1_w, Wx1_b, Wx2_w, Wx2_b, ln_g, ln_b):
    src = edge_index[0].astype(jnp.int32)
    dst = edge_index[1].astype(jnp.int32)
    WiaT = We1_w[:, :H].T
    WibT = We1_w[:, H:2 * H].T
    wr_row = We1_w[:, 2 * H:2 * H + 1].T
    WeET = We1_w[:, 2 * H + 1:].T
    A, Bm = _tc_pre(h, WiaT, WibT, We1_b.reshape(1, H))
    G, r2v, dxv, dyv, dzv = _sc_gather(A, Bm, pos.reshape(-1), src, dst)
    m, gs = _tc_edge(G, edge_attr, r2v.reshape(E, 1), WeET, wr_row,
                     We2_w.T, We2_b.reshape(1, H), Wx1_w.T, Wx1_b.reshape(1, H),
                     Wx2_w, Wx2_b.reshape(1, 1))
    agg2, dp2 = _sc_scatter(m, gs.reshape(E), dxv, dyv, dzv, dst)
    h_out, pos_out = _tc_node(h, agg2, dp2.reshape(2, NP, 3), pos, Wh1_w[:, :H].T, Wh1_w[:, H:].T,
                              Wh1_b.reshape(1, H), Wh2_w.T, Wh2_b.reshape(1, H),
                              ln_g.reshape(1, H), ln_b.reshape(1, H))
    return (h_out, pos_out)


# speedup vs baseline: 3.4535x; 1.1597x over previous
import functools

import jax
import jax.numpy as jnp
from jax import lax
from jax.experimental import pallas as pl
from jax.experimental.pallas import tpu as pltpu
from jax.experimental.pallas import tpu_sc as plsc

H = 128
N = 10000
E = 320000
L = 16
CH = 128
NW = 32
NCH = E // CH
NP = 10240
RB = 128
NB = 1000
BE = 512

_mesh = plsc.VectorSubcoreMesh(core_axis_name="c", subcore_axis_name="s")


def _silu(x):
    return x * jax.nn.sigmoid(x)


def _tc_pre(h, WiaT, WibT, be1):
    def body(h_r, wa_r, wb_r, b_r, A_r, B_r):
        hb = h_r[...]
        A_r[...] = jnp.dot(hb, wa_r[...], preferred_element_type=jnp.float32) + b_r[...]
        B_r[...] = jnp.dot(hb, wb_r[...], preferred_element_type=jnp.float32)

    return pl.pallas_call(
        body,
        grid=(N // NB,),
        in_specs=[
            pl.BlockSpec((NB, H), lambda i: (i, 0)),
            pl.BlockSpec((H, H), lambda i: (0, 0)),
            pl.BlockSpec((H, H), lambda i: (0, 0)),
            pl.BlockSpec((1, H), lambda i: (0, 0)),
        ],
        out_specs=[
            pl.BlockSpec((NB, H), lambda i: (i, 0)),
            pl.BlockSpec((NB, H), lambda i: (i, 0)),
        ],
        out_shape=[jax.ShapeDtypeStruct((N, H), jnp.float32)] * 2,
    )(h, WiaT, WibT, be1)


def _sc_gather(A, B, pos, eic):
    QR = CH // 4

    @functools.partial(
        pl.kernel,
        out_type=[
            jax.ShapeDtypeStruct((E, H), jnp.float32),
            jax.ShapeDtypeStruct((E,), jnp.float32),
            jax.ShapeDtypeStruct((NCH, 3, CH), jnp.float32),
        ],
        mesh=_mesh,
        compiler_params=pltpu.CompilerParams(needs_layout_passes=False),
        scratch_types=[
            pltpu.VMEM((3 * N,), jnp.float32),
            pltpu.VMEM((2, CH), jnp.int32),
            pltpu.VMEM((2, CH), jnp.int32),
            pltpu.VMEM((CH, H), jnp.float32),
            pltpu.VMEM((CH, H), jnp.float32),
            pltpu.VMEM((CH, H), jnp.float32),
            pltpu.VMEM((CH, H), jnp.float32),
            pltpu.VMEM((CH,), jnp.float32),
            pltpu.VMEM((CH,), jnp.float32),
            pltpu.VMEM((3, CH), jnp.float32),
            pltpu.VMEM((3, CH), jnp.float32),
            pltpu.SemaphoreType.DMA,
            pltpu.SemaphoreType.DMA,
            pltpu.SemaphoreType.DMA,
            pltpu.SemaphoreType.DMA,
            pltpu.SemaphoreType.DMA,
            pltpu.SemaphoreType.DMA,
        ],
    )
    def k(A_h, B_h, pos_h, ei_h, G_h, r2_h, d3_h,
          posv, ib0, ib1, bA0, bA1, bB0, bB1, r2b0, r2b1, d3b0, d3b1,
          sA0, sA1, sB0, sB1, sW0, sW1):
        cid = lax.axis_index("c")
        sid = lax.axis_index("s")
        wid = sid * 2 + cid
        nk = 78 + jnp.where(wid < NCH - 78 * NW, 1, 0)
        pltpu.sync_copy(pos_h, posv)
        slots = ((ib0, bA0, bB0, r2b0, d3b0, sA0, sB0, sW0),
                 (ib1, bA1, bB1, r2b1, d3b1, sA1, sB1, sW1))

        def fetch(c, slot):
            ib, bA, bB, _, _, sA, sB, _ = slots[slot]
            g = wid + c * NW
            pltpu.sync_copy(ei_h.at[g], ib)
            pltpu.async_copy(A_h.at[ib.at[0]], bA, sA)
            pltpu.async_copy(B_h.at[ib.at[1]], bB, sB)

        def process(c, slot, prefetch):
            ib, bA, bB, r2b, d3b, sA, sB, sW = slots[slot]
            g = wid + c * NW
            base = g * CH
            for gi in range(CH // L):
                sl = pl.ds(gi * L, L)
                di = ib[0, sl] * 3
                si = ib[1, sl] * 3
                ddx = plsc.load_gather(posv, [di]) - plsc.load_gather(posv, [si])
                ddy = plsc.load_gather(posv, [di + 1]) - plsc.load_gather(posv, [si + 1])
                ddz = plsc.load_gather(posv, [di + 2]) - plsc.load_gather(posv, [si + 2])
                d3b[0, sl] = ddx
                d3b[1, sl] = ddy
                d3b[2, sl] = ddz
                r2b[sl] = ddx * ddx + ddy * ddy + ddz * ddz
            pltpu.make_async_copy(A_h.at[ib.at[0]], bA, sA).wait()
            pltpu.make_async_copy(B_h.at[ib.at[1]], bB, sB).wait()
            for q in range(4):
                def row(r, cc):
                    for qq in range(H // L):
                        s2 = pl.ds(qq * L, L)
                        bA[r, s2] = bA[r, s2] + bB[r, s2]
                    return cc

                lax.fori_loop(q * QR, (q + 1) * QR, row, 0)
                pltpu.async_copy(bA.at[pl.ds(q * QR, QR)],
                                 G_h.at[pl.ds(base + q * QR, QR)], sW)
            pltpu.sync_copy(r2b, r2_h.at[pl.ds(base, CH)])
            pltpu.sync_copy(d3b, d3_h.at[g])
            if prefetch:
                @pl.when(c + 2 < nk)
                def _():
                    for q in range(4):
                        pltpu.make_async_copy(
                            bA.at[pl.ds(q * QR, QR)],
                            G_h.at[pl.ds(base + q * QR, QR)], sW).wait()
                    fetch(c + 2, slot)

        fetch(0, 0)
        fetch(1, 1)

        def body(t, carry):
            process(2 * t, 0, True)
            process(2 * t + 1, 1, True)
            return carry

        lax.fori_loop(0, 39, body, 0)

        @pl.when(wid < NCH - 78 * NW)
        def _():
            process(78, 0, False)

        for q in range(4):
            pltpu.make_async_copy(bA0.at[pl.ds(q * QR, QR)],
                                  G_h.at[pl.ds(q * QR, QR)], sW0).wait()
            pltpu.make_async_copy(bA1.at[pl.ds(q * QR, QR)],
                                  G_h.at[pl.ds(q * QR, QR)], sW1).wait()

    return k(A, B, pos, eic)


def _tc_edge(G, ea, r2c, WeET, wr_row, We2T, be2, Wx1T, bx1, wx2_row, bx2):
    def body(G_r, ea_r, r2_r, weet_r, wr_r, we2t_r, be2_r, wx1t_r, bx1_r,
             wx2_r, bx2_r, m_r, gs_r):
        r2 = r2_r[...]
        pre1 = (G_r[...] + r2 * wr_r[...]
                + jnp.dot(ea_r[...], weet_r[...], preferred_element_type=jnp.float32))
        l1 = _silu(pre1)
        mm = _silu(jnp.dot(l1, we2t_r[...], preferred_element_type=jnp.float32) + be2_r[...])
        m_r[...] = mm
        g1 = _silu(jnp.dot(mm, wx1t_r[...], preferred_element_type=jnp.float32) + bx1_r[...])
        gate = jnp.tanh(jnp.sum(g1 * wx2_r[...], axis=1, keepdims=True) + bx2_r[...])
        gs_r[...] = gate / (r2 + 1.0)

    return pl.pallas_call(
        body,
        grid=(E // BE,),
        in_specs=[
            pl.BlockSpec((BE, H), lambda i: (i, 0)),
            pl.BlockSpec((BE, 16), lambda i: (i, 0)),
            pl.BlockSpec((BE, 1), lambda i: (i, 0)),
            pl.BlockSpec((16, H), lambda i: (0, 0)),
            pl.BlockSpec((1, H), lambda i: (0, 0)),
            pl.BlockSpec((H, H), lambda i: (0, 0)),
            pl.BlockSpec((1, H), lambda i: (0, 0)),
            pl.BlockSpec((H, H), lambda i: (0, 0)),
            pl.BlockSpec((1, H), lambda i: (0, 0)),
            pl.BlockSpec((1, H), lambda i: (0, 0)),
            pl.BlockSpec((1, 1), lambda i: (0, 0)),
        ],
        out_specs=[
            pl.BlockSpec((BE, H), lambda i: (i, 0)),
            pl.BlockSpec((BE, 1), lambda i: (i, 0)),
        ],
        out_shape=[
            jax.ShapeDtypeStruct((E, H), jnp.float32),
            jax.ShapeDtypeStruct((E, 1), jnp.float32),
        ],
    )(G, ea, r2c, WeET, wr_row, We2T, be2, Wx1T, bx1, wx2_row, bx2)


def _sc_scatter(m, gs, d3, dst):
    @functools.partial(
        pl.kernel,
        out_type=[
            jax.ShapeDtypeStruct((2, NP, H), jnp.float32),
            jax.ShapeDtypeStruct((2, 3 * NP), jnp.float32),
        ],
        mesh=_mesh,
        compiler_params=pltpu.CompilerParams(needs_layout_passes=False),
        scratch_types=[
            pltpu.VMEM_SHARED((NP, H), jnp.float32),
            pltpu.VMEM_SHARED((3 * NP,), jnp.float32),
            pltpu.VMEM((CH,), jnp.int32),
            pltpu.VMEM((CH, H), jnp.float32),
            pltpu.VMEM((CH,), jnp.float32),
            pltpu.VMEM((3, CH), jnp.float32),
            pltpu.VMEM((CH,), jnp.float32),
            pltpu.VMEM((CH,), jnp.float32),
            pltpu.VMEM((CH,), jnp.float32),
            pltpu.VMEM((CH,), jnp.int32),
            pltpu.VMEM((CH,), jnp.int32),
            pltpu.VMEM((CH,), jnp.int32),
            pltpu.VMEM((3 * NP // 16,), jnp.float32),
        ],
    )
    def k(m_h, gs_h, d3_h, dst_h, agg_h, dp_h,
          agg_s, dp_s, dbuf, mbuf, gsbuf, d3c,
          cxb, cyb, czb, ixb, iyb, izb, dpb):
        cid = lax.axis_index("c")
        sid = lax.axis_index("s")
        wid = sid * 2 + cid
        zv = jnp.zeros((L,), jnp.float32)
        DPW = 3 * NP // 16

        def zrow(r, c):
            for q in range(H // L):
                mbuf[r, pl.ds(q * L, L)] = zv
            return c

        lax.fori_loop(0, CH, zrow, 0)

        def zdp(r, c):
            dpb[pl.ds(r * L, L)] = zv
            return c

        lax.fori_loop(0, DPW // L, zdp, 0)
        for j in range(5):
            pltpu.sync_copy(mbuf, agg_s.at[pl.ds(sid * 640 + j * RB, RB)])
        pltpu.sync_copy(dpb, dp_s.at[pl.ds(sid * DPW, DPW)])
        plsc.subcore_barrier()
        nk = 78 + jnp.where(wid < NCH - 78 * NW, 1, 0)

        def body(kk, c):
            g = wid + kk * NW
            base = g * CH
            pltpu.sync_copy(dst_h.at[pl.ds(base, CH)], dbuf)
            pltpu.sync_copy(m_h.at[pl.ds(base, CH)], mbuf)
            pltpu.sync_copy(gs_h.at[pl.ds(base, CH)], gsbuf)
            pltpu.sync_copy(d3_h.at[g], d3c)
            for gi in range(CH // L):
                sl = pl.ds(gi * L, L)
                gsv = gsbuf[sl]
                i3 = dbuf[sl] * 3
                ixb[sl] = i3
                iyb[sl] = i3 + 1
                izb[sl] = i3 + 2
                cxb[sl] = d3c[0, sl] * gsv
                cyb[sl] = d3c[1, sl] * gsv
                czb[sl] = d3c[2, sl] * gsv
            pltpu.sync_copy(mbuf, agg_s.at[dbuf], add=True)
            pltpu.sync_copy(cxb, dp_s.at[ixb], add=True)
            pltpu.sync_copy(cyb, dp_s.at[iyb], add=True)
            pltpu.sync_copy(czb, dp_s.at[izb], add=True)
            return c

        lax.fori_loop(0, nk, body, 0)
        plsc.subcore_barrier()
        for j in range(5):
            r0 = sid * 640 + j * RB
            pltpu.sync_copy(agg_s.at[pl.ds(r0, RB)], mbuf)
            pltpu.sync_copy(mbuf, agg_h.at[cid, pl.ds(r0, RB)])
        pltpu.sync_copy(dp_s.at[pl.ds(sid * DPW, DPW)], dpb)
        pltpu.sync_copy(dpb, dp_h.at[cid, pl.ds(sid * DPW, DPW)])

    return k(m, gs, d3, dst)


def _tc_node(h, agg2, dp2, pos, Wh1aT, Wh1bT, bh1, Wh2T, bh2, g, b):
    def body(h_r, agg_r, dp_r, pos_r, wa_r, wb_r, b1_r, w2_r, b2_r, g_r, be_r,
             ho_r, po_r):
        hb = h_r[...]
        a3 = agg_r[...]
        agg = a3[0] + a3[1]
        t = _silu(jnp.dot(hb, wa_r[...], preferred_element_type=jnp.float32)
                  + jnp.dot(agg, wb_r[...], preferred_element_type=jnp.float32)
                  + b1_r[...])
        dh = jnp.dot(t, w2_r[...], preferred_element_type=jnp.float32) + b2_r[...]
        x = hb + dh
        mu = jnp.mean(x, axis=1, keepdims=True)
        var = jnp.mean((x - mu) ** 2, axis=1, keepdims=True)
        ho_r[...] = (x - mu) * lax.rsqrt(var + 1e-5) * g_r[...] + be_r[...]
        d3 = dp_r[...]
        po_r[...] = pos_r[...] + d3[0] + d3[1]

    return pl.pallas_call(
        body,
        grid=(N // NB,),
        in_specs=[
            pl.BlockSpec((NB, H), lambda i: (i, 0)),
            pl.BlockSpec((2, NB, H), lambda i: (0, i, 0)),
            pl.BlockSpec((2, NB, 3), lambda i: (0, i, 0)),
            pl.BlockSpec((NB, 3), lambda i: (i, 0)),
            pl.BlockSpec((H, H), lambda i: (0, 0)),
            pl.BlockSpec((H, H), lambda i: (0, 0)),
            pl.BlockSpec((1, H), lambda i: (0, 0)),
            pl.BlockSpec((H, H), lambda i: (0, 0)),
            pl.BlockSpec((1, H), lambda i: (0, 0)),
            pl.BlockSpec((1, H), lambda i: (0, 0)),
            pl.BlockSpec((1, H), lambda i: (0, 0)),
        ],
        out_specs=[
            pl.BlockSpec((NB, H), lambda i: (i, 0)),
            pl.BlockSpec((NB, 3), lambda i: (i, 0)),
        ],
        out_shape=[
            jax.ShapeDtypeStruct((N, H), jnp.float32),
            jax.ShapeDtypeStruct((N, 3), jnp.float32),
        ],
    )(h, agg2, dp2, pos, Wh1aT, Wh1bT, bh1, Wh2T, bh2, g, b)


def kernel(h, pos, edge_index, edge_attr, We1_w, We1_b, We2_w, We2_b,
           Wh1_w, Wh1_b, Wh2_w, Wh2_b, Wx1_w, Wx1_b, Wx2_w, Wx2_b, ln_g, ln_b):
    src = edge_index[0].astype(jnp.int32)
    dst = edge_index[1].astype(jnp.int32)
    WiaT = We1_w[:, :H].T
    WibT = We1_w[:, H:2 * H].T
    wr_row = We1_w[:, 2 * H:2 * H + 1].T
    WeET = We1_w[:, 2 * H + 1:].T
    A, Bm = _tc_pre(h, WiaT, WibT, We1_b.reshape(1, H))
    eic = jnp.stack([dst, src]).reshape(2, NCH, CH).transpose(1, 0, 2)
    G, r2v, d3v = _sc_gather(A, Bm, pos.reshape(-1), eic)
    m, gs = _tc_edge(G, edge_attr, r2v.reshape(E, 1), WeET, wr_row,
                     We2_w.T, We2_b.reshape(1, H), Wx1_w.T, Wx1_b.reshape(1, H),
                     Wx2_w, Wx2_b.reshape(1, 1))
    agg2, dp2 = _sc_scatter(m, gs.reshape(E), d3v, dst)
    h_out, pos_out = _tc_node(h, agg2, dp2.reshape(2, NP, 3), pos, Wh1_w[:, :H].T, Wh1_w[:, H:].T,
                              Wh1_b.reshape(1, H), Wh2_w.T, Wh2_b.reshape(1, H),
                              ln_g.reshape(1, H), ln_b.reshape(1, H))
    return (h_out, pos_out)


# speedup vs baseline: 3.9795x; 1.1523x over previous
import functools

import jax
import jax.numpy as jnp
from jax import lax
from jax.experimental import pallas as pl
from jax.experimental.pallas import tpu as pltpu
from jax.experimental.pallas import tpu_sc as plsc

H = 128
N = 10000
E = 320000
L = 16
CH = 128
NW = 32
NCH = E // CH
NP = 10240
RB = 128
NB = 1000
BE = 512

_mesh = plsc.VectorSubcoreMesh(core_axis_name="c", subcore_axis_name="s")


def _silu(x):
    return x * jax.nn.sigmoid(x)


def _tc_pre(h, WiaT, WibT, be1):
    def body(h_r, wa_r, wb_r, b_r, A_r, B_r):
        hb = h_r[...]
        A_r[...] = jnp.dot(hb, wa_r[...], preferred_element_type=jnp.float32) + b_r[...]
        B_r[...] = jnp.dot(hb, wb_r[...], preferred_element_type=jnp.float32)

    return pl.pallas_call(
        body,
        grid=(N // NB,),
        in_specs=[
            pl.BlockSpec((NB, H), lambda i: (i, 0)),
            pl.BlockSpec((H, H), lambda i: (0, 0)),
            pl.BlockSpec((H, H), lambda i: (0, 0)),
            pl.BlockSpec((1, H), lambda i: (0, 0)),
        ],
        out_specs=[
            pl.BlockSpec((NB, H), lambda i: (i, 0)),
            pl.BlockSpec((NB, H), lambda i: (i, 0)),
        ],
        out_shape=[jax.ShapeDtypeStruct((N, H), jnp.float32)] * 2,
    )(h, WiaT, WibT, be1)


def _sc_gather(A, B, pos, eic):
    QR = CH // 4

    @functools.partial(
        pl.kernel,
        out_type=[
            jax.ShapeDtypeStruct((E, H), jnp.float32),
            jax.ShapeDtypeStruct((E,), jnp.float32),
            jax.ShapeDtypeStruct((NCH, 3, CH), jnp.float32),
        ],
        mesh=_mesh,
        compiler_params=pltpu.CompilerParams(needs_layout_passes=False),
        scratch_types=[
            pltpu.VMEM((3 * N,), jnp.float32),
            pltpu.VMEM((2, CH), jnp.int32),
            pltpu.VMEM((2, CH), jnp.int32),
            pltpu.VMEM((CH, H), jnp.float32),
            pltpu.VMEM((CH, H), jnp.float32),
            pltpu.VMEM((CH, H), jnp.float32),
            pltpu.VMEM((CH, H), jnp.float32),
            pltpu.VMEM((CH,), jnp.float32),
            pltpu.VMEM((CH,), jnp.float32),
            pltpu.VMEM((3, CH), jnp.float32),
            pltpu.VMEM((3, CH), jnp.float32),
            pltpu.SemaphoreType.DMA,
            pltpu.SemaphoreType.DMA,
            pltpu.SemaphoreType.DMA,
            pltpu.SemaphoreType.DMA,
            pltpu.SemaphoreType.DMA,
            pltpu.SemaphoreType.DMA,
        ],
    )
    def k(A_h, B_h, pos_h, ei_h, G_h, r2_h, d3_h,
          posv, ib0, ib1, bA0, bA1, bB0, bB1, r2b0, r2b1, d3b0, d3b1,
          sA0, sA1, sB0, sB1, sW0, sW1):
        cid = lax.axis_index("c")
        sid = lax.axis_index("s")
        wid = sid * 2 + cid
        nk = 78 + jnp.where(wid < NCH - 78 * NW, 1, 0)
        pltpu.sync_copy(pos_h, posv)
        slots = ((ib0, bA0, bB0, r2b0, d3b0, sA0, sB0, sW0),
                 (ib1, bA1, bB1, r2b1, d3b1, sA1, sB1, sW1))

        def fetch(c, slot):
            ib, bA, bB, _, _, sA, sB, _ = slots[slot]
            g = wid + c * NW
            pltpu.sync_copy(ei_h.at[g], ib)
            pltpu.async_copy(A_h.at[ib.at[0]], bA, sA)
            pltpu.async_copy(B_h.at[ib.at[1]], bB, sB)

        def process(c, slot, prefetch):
            ib, bA, bB, r2b, d3b, sA, sB, sW = slots[slot]
            g = wid + c * NW
            base = g * CH
            for gi in range(CH // L):
                sl = pl.ds(gi * L, L)
                di = ib[0, sl] * 3
                si = ib[1, sl] * 3
                ddx = plsc.load_gather(posv, [di]) - plsc.load_gather(posv, [si])
                ddy = plsc.load_gather(posv, [di + 1]) - plsc.load_gather(posv, [si + 1])
                ddz = plsc.load_gather(posv, [di + 2]) - plsc.load_gather(posv, [si + 2])
                d3b[0, sl] = ddx
                d3b[1, sl] = ddy
                d3b[2, sl] = ddz
                r2b[sl] = ddx * ddx + ddy * ddy + ddz * ddz
            pltpu.make_async_copy(A_h.at[ib.at[0]], bA, sA).wait()
            pltpu.make_async_copy(B_h.at[ib.at[1]], bB, sB).wait()
            for q in range(4):
                def row(r, cc):
                    for qq in range(H // L):
                        s2 = pl.ds(qq * L, L)
                        bA[r, s2] = bA[r, s2] + bB[r, s2]
                    return cc

                lax.fori_loop(q * QR, (q + 1) * QR, row, 0)
                pltpu.async_copy(bA.at[pl.ds(q * QR, QR)],
                                 G_h.at[pl.ds(base + q * QR, QR)], sW)
            pltpu.sync_copy(r2b, r2_h.at[pl.ds(base, CH)])
            pltpu.sync_copy(d3b, d3_h.at[g])
            if prefetch:
                @pl.when(c + 2 < nk)
                def _():
                    for q in range(4):
                        pltpu.make_async_copy(
                            bA.at[pl.ds(q * QR, QR)],
                            G_h.at[pl.ds(base + q * QR, QR)], sW).wait()
                    fetch(c + 2, slot)

        fetch(0, 0)
        fetch(1, 1)

        def body(t, carry):
            process(2 * t, 0, True)
            process(2 * t + 1, 1, True)
            return carry

        lax.fori_loop(0, 39, body, 0)

        @pl.when(wid < NCH - 78 * NW)
        def _():
            process(78, 0, False)

        for q in range(4):
            pltpu.make_async_copy(bA0.at[pl.ds(q * QR, QR)],
                                  G_h.at[pl.ds(q * QR, QR)], sW0).wait()
            pltpu.make_async_copy(bA1.at[pl.ds(q * QR, QR)],
                                  G_h.at[pl.ds(q * QR, QR)], sW1).wait()

    return k(A, B, pos, eic)


def _tc_edge(G, ea, r2c, WeET, wr_row, We2T, be2, Wx1T, bx1, wx2_row, bx2):
    def body(G_r, ea_r, r2_r, weet_r, wr_r, we2t_r, be2_r, wx1t_r, bx1_r,
             wx2_r, bx2_r, m_r, gs_r):
        r2 = r2_r[...]
        pre1 = (G_r[...] + r2 * wr_r[...]
                + jnp.dot(ea_r[...], weet_r[...], preferred_element_type=jnp.float32))
        l1 = _silu(pre1)
        mm = _silu(jnp.dot(l1, we2t_r[...], preferred_element_type=jnp.float32) + be2_r[...])
        m_r[...] = mm
        g1 = _silu(jnp.dot(mm, wx1t_r[...], preferred_element_type=jnp.float32) + bx1_r[...])
        gate = jnp.tanh(jnp.sum(g1 * wx2_r[...], axis=1, keepdims=True) + bx2_r[...])
        gs_r[...] = gate / (r2 + 1.0)

    return pl.pallas_call(
        body,
        grid=(E // BE,),
        in_specs=[
            pl.BlockSpec((BE, H), lambda i: (i, 0)),
            pl.BlockSpec((BE, 16), lambda i: (i, 0)),
            pl.BlockSpec((BE, 1), lambda i: (i, 0)),
            pl.BlockSpec((16, H), lambda i: (0, 0)),
            pl.BlockSpec((1, H), lambda i: (0, 0)),
            pl.BlockSpec((H, H), lambda i: (0, 0)),
            pl.BlockSpec((1, H), lambda i: (0, 0)),
            pl.BlockSpec((H, H), lambda i: (0, 0)),
            pl.BlockSpec((1, H), lambda i: (0, 0)),
            pl.BlockSpec((1, H), lambda i: (0, 0)),
            pl.BlockSpec((1, 1), lambda i: (0, 0)),
        ],
        out_specs=[
            pl.BlockSpec((BE, H), lambda i: (i, 0)),
            pl.BlockSpec((BE, 1), lambda i: (i, 0)),
        ],
        out_shape=[
            jax.ShapeDtypeStruct((E, H), jnp.float32),
            jax.ShapeDtypeStruct((E, 1), jnp.float32),
        ],
    )(G, ea, r2c, WeET, wr_row, We2T, be2, Wx1T, bx1, wx2_row, bx2)


def _sc_scatter(m, gs, d3, dst):
    @functools.partial(
        pl.kernel,
        out_type=[
            jax.ShapeDtypeStruct((2, NP, H), jnp.float32),
            jax.ShapeDtypeStruct((2, 3 * NP), jnp.float32),
        ],
        mesh=_mesh,
        compiler_params=pltpu.CompilerParams(needs_layout_passes=False),
        scratch_types=[
            pltpu.VMEM_SHARED((NP, H), jnp.float32),
            pltpu.VMEM_SHARED((3 * NP,), jnp.float32),
            pltpu.VMEM((CH,), jnp.int32),
            pltpu.VMEM((CH,), jnp.int32),
            pltpu.VMEM((CH, H), jnp.float32),
            pltpu.VMEM((CH, H), jnp.float32),
            pltpu.VMEM((CH,), jnp.float32),
            pltpu.VMEM((CH,), jnp.float32),
            pltpu.VMEM((3, CH), jnp.float32),
            pltpu.VMEM((3, CH), jnp.float32),
            pltpu.VMEM((3, CH), jnp.float32),
            pltpu.VMEM((3, CH), jnp.float32),
            pltpu.VMEM((3, CH), jnp.int32),
            pltpu.VMEM((3, CH), jnp.int32),
            pltpu.VMEM((3 * NP // 16,), jnp.float32),
            pltpu.SemaphoreType.DMA,
            pltpu.SemaphoreType.DMA,
            pltpu.SemaphoreType.DMA,
            pltpu.SemaphoreType.DMA,
        ],
    )
    def k(m_h, gs_h, d3_h, dst_h, agg_h, dp_h,
          agg_s, dp_s, dbuf0, dbuf1, mbuf0, mbuf1, gsb0, gsb1,
          d3c0, d3c1, cb0, cb1, ixb0, ixb1, dpb, sI0, sI1, sS0, sS1):
        cid = lax.axis_index("c")
        sid = lax.axis_index("s")
        wid = sid * 2 + cid
        nk = 78 + jnp.where(wid < NCH - 78 * NW, 1, 0)
        zv = jnp.zeros((L,), jnp.float32)
        DPW = 3 * NP // 16
        slots = ((dbuf0, mbuf0, gsb0, d3c0, cb0, ixb0, sI0, sS0),
                 (dbuf1, mbuf1, gsb1, d3c1, cb1, ixb1, sI1, sS1))

        def zrow(r, c):
            for q in range(H // L):
                mbuf0[r, pl.ds(q * L, L)] = zv
            return c

        lax.fori_loop(0, CH, zrow, 0)

        def zdp(r, c):
            dpb[pl.ds(r * L, L)] = zv
            return c

        lax.fori_loop(0, DPW // L, zdp, 0)
        for j in range(5):
            pltpu.sync_copy(mbuf0, agg_s.at[pl.ds(sid * 640 + j * RB, RB)])
        pltpu.sync_copy(dpb, dp_s.at[pl.ds(sid * DPW, DPW)])
        plsc.subcore_barrier()

        def fetch(c, slot):
            dbuf, mbuf, gsb, d3c, _, _, sI, _ = slots[slot]
            g = wid + c * NW
            base = g * CH
            pltpu.async_copy(dst_h.at[pl.ds(base, CH)], dbuf, sI)
            pltpu.async_copy(m_h.at[pl.ds(base, CH)], mbuf, sI)
            pltpu.async_copy(gs_h.at[pl.ds(base, CH)], gsb, sI)
            pltpu.async_copy(d3_h.at[g], d3c, sI)

        def process(c, slot, prefetch):
            dbuf, mbuf, gsb, d3c, cb, ixb, sI, sS = slots[slot]
            g = wid + c * NW
            base = g * CH
            pltpu.make_async_copy(dst_h.at[pl.ds(base, CH)], dbuf, sI).wait()
            pltpu.make_async_copy(m_h.at[pl.ds(base, CH)], mbuf, sI).wait()
            pltpu.make_async_copy(gs_h.at[pl.ds(base, CH)], gsb, sI).wait()
            pltpu.make_async_copy(d3_h.at[g], d3c, sI).wait()
            for gi in range(CH // L):
                sl = pl.ds(gi * L, L)
                gsv = gsb[sl]
                i3 = dbuf[sl] * 3
                ixb[0, sl] = i3
                ixb[1, sl] = i3 + 1
                ixb[2, sl] = i3 + 2
                cb[0, sl] = d3c[0, sl] * gsv
                cb[1, sl] = d3c[1, sl] * gsv
                cb[2, sl] = d3c[2, sl] * gsv
            pltpu.async_copy(mbuf, agg_s.at[dbuf], sS, add=True)
            pltpu.async_copy(cb.at[0], dp_s.at[ixb.at[0]], sS, add=True)
            pltpu.async_copy(cb.at[1], dp_s.at[ixb.at[1]], sS, add=True)
            pltpu.async_copy(cb.at[2], dp_s.at[ixb.at[2]], sS, add=True)
            if prefetch:
                @pl.when(c + 2 < nk)
                def _():
                    pltpu.make_async_copy(mbuf, agg_s.at[dbuf], sS).wait()
                    pltpu.make_async_copy(cb.at[0], dp_s.at[ixb.at[0]], sS).wait()
                    pltpu.make_async_copy(cb.at[1], dp_s.at[ixb.at[1]], sS).wait()
                    pltpu.make_async_copy(cb.at[2], dp_s.at[ixb.at[2]], sS).wait()
                    fetch(c + 2, slot)

        fetch(0, 0)
        fetch(1, 1)

        def body(t, carry):
            process(2 * t, 0, True)
            process(2 * t + 1, 1, True)
            return carry

        lax.fori_loop(0, 39, body, 0)

        @pl.when(wid < NCH - 78 * NW)
        def _():
            process(78, 0, False)

        for slot in range(2):
            dbuf, mbuf, _, _, cb, ixb, _, sS = slots[slot]
            pltpu.make_async_copy(mbuf, agg_s.at[dbuf], sS).wait()
            pltpu.make_async_copy(cb.at[0], dp_s.at[ixb.at[0]], sS).wait()
            pltpu.make_async_copy(cb.at[1], dp_s.at[ixb.at[1]], sS).wait()
            pltpu.make_async_copy(cb.at[2], dp_s.at[ixb.at[2]], sS).wait()
        plsc.subcore_barrier()
        for j in range(5):
            r0 = sid * 640 + j * RB
            pltpu.sync_copy(agg_s.at[pl.ds(r0, RB)], mbuf0)
            pltpu.sync_copy(mbuf0, agg_h.at[cid, pl.ds(r0, RB)])
        pltpu.sync_copy(dp_s.at[pl.ds(sid * DPW, DPW)], dpb)
        pltpu.sync_copy(dpb, dp_h.at[cid, pl.ds(sid * DPW, DPW)])

    return k(m, gs, d3, dst)


def _tc_node(h, agg2, dp2, pos, Wh1aT, Wh1bT, bh1, Wh2T, bh2, g, b):
    def body(h_r, agg_r, dp_r, pos_r, wa_r, wb_r, b1_r, w2_r, b2_r, g_r, be_r,
             ho_r, po_r):
        hb = h_r[...]
        a3 = agg_r[...]
        agg = a3[0] + a3[1]
        t = _silu(jnp.dot(hb, wa_r[...], preferred_element_type=jnp.float32)
                  + jnp.dot(agg, wb_r[...], preferred_element_type=jnp.float32)
                  + b1_r[...])
        dh = jnp.dot(t, w2_r[...], preferred_element_type=jnp.float32) + b2_r[...]
        x = hb + dh
        mu = jnp.mean(x, axis=1, keepdims=True)
        var = jnp.mean((x - mu) ** 2, axis=1, keepdims=True)
        ho_r[...] = (x - mu) * lax.rsqrt(var + 1e-5) * g_r[...] + be_r[...]
        d3 = dp_r[...]
        po_r[...] = pos_r[...] + d3[0] + d3[1]

    return pl.pallas_call(
        body,
        grid=(N // NB,),
        in_specs=[
            pl.BlockSpec((NB, H), lambda i: (i, 0)),
            pl.BlockSpec((2, NB, H), lambda i: (0, i, 0)),
            pl.BlockSpec((2, NB, 3), lambda i: (0, i, 0)),
            pl.BlockSpec((NB, 3), lambda i: (i, 0)),
            pl.BlockSpec((H, H), lambda i: (0, 0)),
            pl.BlockSpec((H, H), lambda i: (0, 0)),
            pl.BlockSpec((1, H), lambda i: (0, 0)),
            pl.BlockSpec((H, H), lambda i: (0, 0)),
            pl.BlockSpec((1, H), lambda i: (0, 0)),
            pl.BlockSpec((1, H), lambda i: (0, 0)),
            pl.BlockSpec((1, H), lambda i: (0, 0)),
        ],
        out_specs=[
            pl.BlockSpec((NB, H), lambda i: (i, 0)),
            pl.BlockSpec((NB, 3), lambda i: (i, 0)),
        ],
        out_shape=[
            jax.ShapeDtypeStruct((N, H), jnp.float32),
            jax.ShapeDtypeStruct((N, 3), jnp.float32),
        ],
    )(h, agg2, dp2, pos, Wh1aT, Wh1bT, bh1, Wh2T, bh2, g, b)


def kernel(h, pos, edge_index, edge_attr, We1_w, We1_b, We2_w, We2_b,
           Wh1_w, Wh1_b, Wh2_w, Wh2_b, Wx1_w, Wx1_b, Wx2_w, Wx2_b, ln_g, ln_b):
    src = edge_index[0].astype(jnp.int32)
    dst = edge_index[1].astype(jnp.int32)
    WiaT = We1_w[:, :H].T
    WibT = We1_w[:, H:2 * H].T
    wr_row = We1_w[:, 2 * H:2 * H + 1].T
    WeET = We1_w[:, 2 * H + 1:].T
    A, Bm = _tc_pre(h, WiaT, WibT, We1_b.reshape(1, H))
    eic = jnp.stack([dst, src]).reshape(2, NCH, CH).transpose(1, 0, 2)
    G, r2v, d3v = _sc_gather(A, Bm, pos.reshape(-1), eic)
    m, gs = _tc_edge(G, edge_attr, r2v.reshape(E, 1), WeET, wr_row,
                     We2_w.T, We2_b.reshape(1, H), Wx1_w.T, Wx1_b.reshape(1, H),
                     Wx2_w, Wx2_b.reshape(1, 1))
    agg2, dp2 = _sc_scatter(m, gs.reshape(E), d3v, dst)
    h_out, pos_out = _tc_node(h, agg2, dp2.reshape(2, NP, 3), pos, Wh1_w[:, :H].T, Wh1_w[:, H:].T,
                              Wh1_b.reshape(1, H), Wh2_w.T, Wh2_b.reshape(1, H),
                              ln_g.reshape(1, H), ln_b.reshape(1, H))
    return (h_out, pos_out)


# speedup vs baseline: 5.2343x; 1.3153x over previous
import functools

import jax
import jax.numpy as jnp
from jax import lax
from jax.experimental import pallas as pl
from jax.experimental.pallas import tpu as pltpu
from jax.experimental.pallas import tpu_sc as plsc

H = 128
N = 10000
E = 320000
L = 16
CH = 128
NW = 32
NCH = E // CH
NP = 10240
RB = 128
NB = 1000
BE = 1280

_mesh = plsc.VectorSubcoreMesh(core_axis_name="c", subcore_axis_name="s")


def _silu(x):
    return x * jax.nn.sigmoid(x)


def _tc_pre(h, WiaT, WibT, be1):
    def body(h_r, wa_r, wb_r, b_r, A_r, B_r):
        hb = h_r[...]
        A_r[...] = jnp.dot(hb, wa_r[...], preferred_element_type=jnp.float32) + b_r[...]
        B_r[...] = jnp.dot(hb, wb_r[...], preferred_element_type=jnp.float32)

    return pl.pallas_call(
        body,
        grid=(N // NB,),
        in_specs=[
            pl.BlockSpec((NB, H), lambda i: (i, 0)),
            pl.BlockSpec((H, H), lambda i: (0, 0)),
            pl.BlockSpec((H, H), lambda i: (0, 0)),
            pl.BlockSpec((1, H), lambda i: (0, 0)),
        ],
        out_specs=[
            pl.BlockSpec((NB, H), lambda i: (i, 0)),
            pl.BlockSpec((NB, H), lambda i: (i, 0)),
        ],
        out_shape=[jax.ShapeDtypeStruct((N, H), jnp.float32)] * 2,
    )(h, WiaT, WibT, be1)


def _sc_gather(A, B, pos, eic):
    QR = CH // 4

    @functools.partial(
        pl.kernel,
        out_type=[
            jax.ShapeDtypeStruct((E, H), jnp.float32),
            jax.ShapeDtypeStruct((E,), jnp.float32),
            jax.ShapeDtypeStruct((NCH, 3, CH), jnp.float32),
        ],
        mesh=_mesh,
        compiler_params=pltpu.CompilerParams(needs_layout_passes=False),
        scratch_types=[
            pltpu.VMEM((3 * N,), jnp.float32),
            pltpu.VMEM((2, CH), jnp.int32),
            pltpu.VMEM((2, CH), jnp.int32),
            pltpu.VMEM((CH, H), jnp.float32),
            pltpu.VMEM((CH, H), jnp.float32),
            pltpu.VMEM((CH, H), jnp.float32),
            pltpu.VMEM((CH, H), jnp.float32),
            pltpu.VMEM((CH,), jnp.float32),
            pltpu.VMEM((CH,), jnp.float32),
            pltpu.VMEM((3, CH), jnp.float32),
            pltpu.VMEM((3, CH), jnp.float32),
            pltpu.SemaphoreType.DMA,
            pltpu.SemaphoreType.DMA,
            pltpu.SemaphoreType.DMA,
            pltpu.SemaphoreType.DMA,
            pltpu.SemaphoreType.DMA,
            pltpu.SemaphoreType.DMA,
        ],
    )
    def k(A_h, B_h, pos_h, ei_h, G_h, r2_h, d3_h,
          posv, ib0, ib1, bA0, bA1, bB0, bB1, r2b0, r2b1, d3b0, d3b1,
          sA0, sA1, sB0, sB1, sW0, sW1):
        cid = lax.axis_index("c")
        sid = lax.axis_index("s")
        wid = sid * 2 + cid
        nk = 78 + jnp.where(wid < NCH - 78 * NW, 1, 0)
        pltpu.sync_copy(pos_h, posv)
        slots = ((ib0, bA0, bB0, r2b0, d3b0, sA0, sB0, sW0),
                 (ib1, bA1, bB1, r2b1, d3b1, sA1, sB1, sW1))

        def fetch(c, slot):
            ib, bA, bB, _, _, sA, sB, _ = slots[slot]
            g = wid + c * NW
            pltpu.sync_copy(ei_h.at[g], ib)
            pltpu.async_copy(A_h.at[ib.at[0]], bA, sA)
            pltpu.async_copy(B_h.at[ib.at[1]], bB, sB)

        def process(c, slot, prefetch):
            ib, bA, bB, r2b, d3b, sA, sB, sW = slots[slot]
            g = wid + c * NW
            base = g * CH
            for gi in range(CH // L):
                sl = pl.ds(gi * L, L)
                di = ib[0, sl] * 3
                si = ib[1, sl] * 3
                ddx = plsc.load_gather(posv, [di]) - plsc.load_gather(posv, [si])
                ddy = plsc.load_gather(posv, [di + 1]) - plsc.load_gather(posv, [si + 1])
                ddz = plsc.load_gather(posv, [di + 2]) - plsc.load_gather(posv, [si + 2])
                d3b[0, sl] = ddx
                d3b[1, sl] = ddy
                d3b[2, sl] = ddz
                r2b[sl] = ddx * ddx + ddy * ddy + ddz * ddz
            pltpu.make_async_copy(A_h.at[ib.at[0]], bA, sA).wait()
            pltpu.make_async_copy(B_h.at[ib.at[1]], bB, sB).wait()
            for q in range(4):
                def row(r, cc):
                    for qq in range(H // L):
                        s2 = pl.ds(qq * L, L)
                        bA[r, s2] = bA[r, s2] + bB[r, s2]
                    return cc

                lax.fori_loop(q * QR, (q + 1) * QR, row, 0)
                pltpu.async_copy(bA.at[pl.ds(q * QR, QR)],
                                 G_h.at[pl.ds(base + q * QR, QR)], sW)
            pltpu.sync_copy(r2b, r2_h.at[pl.ds(base, CH)])
            pltpu.sync_copy(d3b, d3_h.at[g])
            if prefetch:
                @pl.when(c + 2 < nk)
                def _():
                    for q in range(4):
                        pltpu.make_async_copy(
                            bA.at[pl.ds(q * QR, QR)],
                            G_h.at[pl.ds(base + q * QR, QR)], sW).wait()
                    fetch(c + 2, slot)

        fetch(0, 0)
        fetch(1, 1)

        def body(t, carry):
            process(2 * t, 0, True)
            process(2 * t + 1, 1, True)
            return carry

        lax.fori_loop(0, 39, body, 0)

        @pl.when(wid < NCH - 78 * NW)
        def _():
            process(78, 0, False)

        for q in range(4):
            pltpu.make_async_copy(bA0.at[pl.ds(q * QR, QR)],
                                  G_h.at[pl.ds(q * QR, QR)], sW0).wait()
            pltpu.make_async_copy(bA1.at[pl.ds(q * QR, QR)],
                                  G_h.at[pl.ds(q * QR, QR)], sW1).wait()

    return k(A, B, pos, eic)


def _tc_edge(G, ea, r2c, WeET, wr_row, We2T, be2, Wx1T, bx1, wx2_row, bx2):
    def body(G_r, ea_r, r2_r, weet_r, wr_r, we2t_r, be2_r, wx1t_r, bx1_r,
             wx2_r, bx2_r, m_r, gs_r):
        r2 = r2_r[...]
        pre1 = (G_r[...] + r2 * wr_r[...]
                + jnp.dot(ea_r[...], weet_r[...], preferred_element_type=jnp.float32))
        l1 = _silu(pre1).astype(jnp.bfloat16)
        mm = _silu(jnp.dot(l1, we2t_r[...], preferred_element_type=jnp.float32) + be2_r[...])
        m_r[...] = mm
        g1 = _silu(jnp.dot(mm.astype(jnp.bfloat16), wx1t_r[...],
                           preferred_element_type=jnp.float32) + bx1_r[...])
        gate = jnp.tanh(jnp.sum(g1 * wx2_r[...], axis=1, keepdims=True) + bx2_r[...])
        gs_r[...] = gate / (r2 + 1.0)

    return pl.pallas_call(
        body,
        grid=(E // BE,),
        in_specs=[
            pl.BlockSpec((BE, H), lambda i: (i, 0)),
            pl.BlockSpec((BE, 16), lambda i: (i, 0)),
            pl.BlockSpec((BE, 1), lambda i: (i, 0)),
            pl.BlockSpec((16, H), lambda i: (0, 0)),
            pl.BlockSpec((1, H), lambda i: (0, 0)),
            pl.BlockSpec((H, H), lambda i: (0, 0)),
            pl.BlockSpec((1, H), lambda i: (0, 0)),
            pl.BlockSpec((H, H), lambda i: (0, 0)),
            pl.BlockSpec((1, H), lambda i: (0, 0)),
            pl.BlockSpec((1, H), lambda i: (0, 0)),
            pl.BlockSpec((1, 1), lambda i: (0, 0)),
        ],
        out_specs=[
            pl.BlockSpec((BE, H), lambda i: (i, 0)),
            pl.BlockSpec((BE, 1), lambda i: (i, 0)),
        ],
        out_shape=[
            jax.ShapeDtypeStruct((E, H), jnp.float32),
            jax.ShapeDtypeStruct((E, 1), jnp.float32),
        ],
    )(G, ea, r2c, WeET, wr_row, We2T, be2, Wx1T, bx1, wx2_row, bx2)


def _sc_scatter(m, gs, d3, dst):
    @functools.partial(
        pl.kernel,
        out_type=[
            jax.ShapeDtypeStruct((2, NP, H), jnp.float32),
            jax.ShapeDtypeStruct((2, 3 * NP), jnp.float32),
        ],
        mesh=_mesh,
        compiler_params=pltpu.CompilerParams(needs_layout_passes=False),
        scratch_types=[
            pltpu.VMEM_SHARED((NP, H), jnp.float32),
            pltpu.VMEM_SHARED((3 * NP,), jnp.float32),
            pltpu.VMEM((CH,), jnp.int32),
            pltpu.VMEM((CH,), jnp.int32),
            pltpu.VMEM((CH, H), jnp.float32),
            pltpu.VMEM((CH, H), jnp.float32),
            pltpu.VMEM((CH,), jnp.float32),
            pltpu.VMEM((CH,), jnp.float32),
            pltpu.VMEM((3, CH), jnp.float32),
            pltpu.VMEM((3, CH), jnp.float32),
            pltpu.VMEM((3, CH), jnp.float32),
            pltpu.VMEM((3, CH), jnp.float32),
            pltpu.VMEM((3, CH), jnp.int32),
            pltpu.VMEM((3, CH), jnp.int32),
            pltpu.VMEM((3 * NP // 16,), jnp.float32),
            pltpu.SemaphoreType.DMA,
            pltpu.SemaphoreType.DMA,
            pltpu.SemaphoreType.DMA,
            pltpu.SemaphoreType.DMA,
        ],
    )
    def k(m_h, gs_h, d3_h, dst_h, agg_h, dp_h,
          agg_s, dp_s, dbuf0, dbuf1, mbuf0, mbuf1, gsb0, gsb1,
          d3c0, d3c1, cb0, cb1, ixb0, ixb1, dpb, sI0, sI1, sS0, sS1):
        cid = lax.axis_index("c")
        sid = lax.axis_index("s")
        wid = sid * 2 + cid
        nk = 78 + jnp.where(wid < NCH - 78 * NW, 1, 0)
        zv = jnp.zeros((L,), jnp.float32)
        DPW = 3 * NP // 16
        slots = ((dbuf0, mbuf0, gsb0, d3c0, cb0, ixb0, sI0, sS0),
                 (dbuf1, mbuf1, gsb1, d3c1, cb1, ixb1, sI1, sS1))

        def zrow(r, c):
            for q in range(H // L):
                mbuf0[r, pl.ds(q * L, L)] = zv
            return c

        lax.fori_loop(0, CH, zrow, 0)

        def zdp(r, c):
            dpb[pl.ds(r * L, L)] = zv
            return c

        lax.fori_loop(0, DPW // L, zdp, 0)
        for j in range(5):
            pltpu.sync_copy(mbuf0, agg_s.at[pl.ds(sid * 640 + j * RB, RB)])
        pltpu.sync_copy(dpb, dp_s.at[pl.ds(sid * DPW, DPW)])
        plsc.subcore_barrier()

        def fetch(c, slot):
            dbuf, mbuf, gsb, d3c, _, _, sI, _ = slots[slot]
            g = wid + c * NW
            base = g * CH
            pltpu.async_copy(dst_h.at[pl.ds(base, CH)], dbuf, sI)
            pltpu.async_copy(m_h.at[pl.ds(base, CH)], mbuf, sI)
            pltpu.async_copy(gs_h.at[pl.ds(base, CH)], gsb, sI)
            pltpu.async_copy(d3_h.at[g], d3c, sI)

        def process(c, slot, prefetch):
            dbuf, mbuf, gsb, d3c, cb, ixb, sI, sS = slots[slot]
            g = wid + c * NW
            base = g * CH
            pltpu.make_async_copy(dst_h.at[pl.ds(base, CH)], dbuf, sI).wait()
            pltpu.make_async_copy(m_h.at[pl.ds(base, CH)], mbuf, sI).wait()
            pltpu.make_async_copy(gs_h.at[pl.ds(base, CH)], gsb, sI).wait()
            pltpu.make_async_copy(d3_h.at[g], d3c, sI).wait()
            for gi in range(CH // L):
                sl = pl.ds(gi * L, L)
                gsv = gsb[sl]
                i3 = dbuf[sl] * 3
                ixb[0, sl] = i3
                ixb[1, sl] = i3 + 1
                ixb[2, sl] = i3 + 2
                cb[0, sl] = d3c[0, sl] * gsv
                cb[1, sl] = d3c[1, sl] * gsv
                cb[2, sl] = d3c[2, sl] * gsv
            pltpu.async_copy(mbuf, agg_s.at[dbuf], sS, add=True)
            pltpu.async_copy(cb.at[0], dp_s.at[ixb.at[0]], sS, add=True)
            pltpu.async_copy(cb.at[1], dp_s.at[ixb.at[1]], sS, add=True)
            pltpu.async_copy(cb.at[2], dp_s.at[ixb.at[2]], sS, add=True)
            if prefetch:
                @pl.when(c + 2 < nk)
                def _():
                    pltpu.make_async_copy(mbuf, agg_s.at[dbuf], sS).wait()
                    pltpu.make_async_copy(cb.at[0], dp_s.at[ixb.at[0]], sS).wait()
                    pltpu.make_async_copy(cb.at[1], dp_s.at[ixb.at[1]], sS).wait()
                    pltpu.make_async_copy(cb.at[2], dp_s.at[ixb.at[2]], sS).wait()
                    fetch(c + 2, slot)

        fetch(0, 0)
        fetch(1, 1)

        def body(t, carry):
            process(2 * t, 0, True)
            process(2 * t + 1, 1, True)
            return carry

        lax.fori_loop(0, 39, body, 0)

        @pl.when(wid < NCH - 78 * NW)
        def _():
            process(78, 0, False)

        for slot in range(2):
            dbuf, mbuf, _, _, cb, ixb, _, sS = slots[slot]
            pltpu.make_async_copy(mbuf, agg_s.at[dbuf], sS).wait()
            pltpu.make_async_copy(cb.at[0], dp_s.at[ixb.at[0]], sS).wait()
            pltpu.make_async_copy(cb.at[1], dp_s.at[ixb.at[1]], sS).wait()
            pltpu.make_async_copy(cb.at[2], dp_s.at[ixb.at[2]], sS).wait()
        plsc.subcore_barrier()
        for j in range(5):
            r0 = sid * 640 + j * RB
            pltpu.sync_copy(agg_s.at[pl.ds(r0, RB)], mbuf0)
            pltpu.sync_copy(mbuf0, agg_h.at[cid, pl.ds(r0, RB)])
        pltpu.sync_copy(dp_s.at[pl.ds(sid * DPW, DPW)], dpb)
        pltpu.sync_copy(dpb, dp_h.at[cid, pl.ds(sid * DPW, DPW)])

    return k(m, gs, d3, dst)


def _tc_node(h, agg2, dp2, pos, Wh1aT, Wh1bT, bh1, Wh2T, bh2, g, b):
    def body(h_r, agg_r, dp_r, pos_r, wa_r, wb_r, b1_r, w2_r, b2_r, g_r, be_r,
             ho_r, po_r):
        hb = h_r[...]
        a3 = agg_r[...]
        agg = a3[0] + a3[1]
        t = _silu(jnp.dot(hb, wa_r[...], preferred_element_type=jnp.float32)
                  + jnp.dot(agg, wb_r[...], preferred_element_type=jnp.float32)
                  + b1_r[...])
        dh = jnp.dot(t, w2_r[...], preferred_element_type=jnp.float32) + b2_r[...]
        x = hb + dh
        mu = jnp.mean(x, axis=1, keepdims=True)
        var = jnp.mean((x - mu) ** 2, axis=1, keepdims=True)
        ho_r[...] = (x - mu) * lax.rsqrt(var + 1e-5) * g_r[...] + be_r[...]
        d3 = dp_r[...]
        po_r[...] = pos_r[...] + d3[0] + d3[1]

    return pl.pallas_call(
        body,
        grid=(N // NB,),
        in_specs=[
            pl.BlockSpec((NB, H), lambda i: (i, 0)),
            pl.BlockSpec((2, NB, H), lambda i: (0, i, 0)),
            pl.BlockSpec((2, NB, 3), lambda i: (0, i, 0)),
            pl.BlockSpec((NB, 3), lambda i: (i, 0)),
            pl.BlockSpec((H, H), lambda i: (0, 0)),
            pl.BlockSpec((H, H), lambda i: (0, 0)),
            pl.BlockSpec((1, H), lambda i: (0, 0)),
            pl.BlockSpec((H, H), lambda i: (0, 0)),
            pl.BlockSpec((1, H), lambda i: (0, 0)),
            pl.BlockSpec((1, H), lambda i: (0, 0)),
            pl.BlockSpec((1, H), lambda i: (0, 0)),
        ],
        out_specs=[
            pl.BlockSpec((NB, H), lambda i: (i, 0)),
            pl.BlockSpec((NB, 3), lambda i: (i, 0)),
        ],
        out_shape=[
            jax.ShapeDtypeStruct((N, H), jnp.float32),
            jax.ShapeDtypeStruct((N, 3), jnp.float32),
        ],
    )(h, agg2, dp2, pos, Wh1aT, Wh1bT, bh1, Wh2T, bh2, g, b)


def kernel(h, pos, edge_index, edge_attr, We1_w, We1_b, We2_w, We2_b,
           Wh1_w, Wh1_b, Wh2_w, Wh2_b, Wx1_w, Wx1_b, Wx2_w, Wx2_b, ln_g, ln_b):
    src = edge_index[0].astype(jnp.int32)
    dst = edge_index[1].astype(jnp.int32)
    WiaT = We1_w[:, :H].T
    WibT = We1_w[:, H:2 * H].T
    wr_row = We1_w[:, 2 * H:2 * H + 1].T
    WeET = We1_w[:, 2 * H + 1:].T
    A, Bm = _tc_pre(h, WiaT, WibT, We1_b.reshape(1, H))
    eic = jnp.stack([dst, src]).reshape(2, NCH, CH).transpose(1, 0, 2)
    G, r2v, d3v = _sc_gather(A, Bm, pos.reshape(-1), eic)
    m, gs = _tc_edge(G, edge_attr.astype(jnp.bfloat16),
                     r2v.reshape(E, 1), WeET.astype(jnp.bfloat16), wr_row,
                     We2_w.T.astype(jnp.bfloat16), We2_b.reshape(1, H),
                     Wx1_w.T.astype(jnp.bfloat16), Wx1_b.reshape(1, H),
                     Wx2_w, Wx2_b.reshape(1, 1))
    agg2, dp2 = _sc_scatter(m, gs.reshape(E), d3v, dst)
    h_out, pos_out = _tc_node(h, agg2, dp2.reshape(2, NP, 3), pos, Wh1_w[:, :H].T, Wh1_w[:, H:].T,
                              Wh1_b.reshape(1, H), Wh2_w.T, Wh2_b.reshape(1, H),
                              ln_g.reshape(1, H), ln_b.reshape(1, H))
    return (h_out, pos_out)


# speedup vs baseline: 5.6901x; 1.0871x over previous
import functools

import jax
import jax.numpy as jnp
from jax import lax
from jax.experimental import pallas as pl
from jax.experimental.pallas import tpu as pltpu
from jax.experimental.pallas import tpu_sc as plsc

H = 128
N = 10000
E = 320000
L = 16
CH = 128
NW = 32
NCH = E // CH
NP = 10240
RB = 128
NB = 1000
BE = 2560

_mesh = plsc.VectorSubcoreMesh(core_axis_name="c", subcore_axis_name="s")


def _silu(x):
    return x * jax.nn.sigmoid(x)


def _tc_pre(h, WiaT, WibT, be1):
    def body(h_r, wa_r, wb_r, b_r, A_r, B_r):
        hb = h_r[...]
        A_r[...] = jnp.dot(hb, wa_r[...], preferred_element_type=jnp.float32) + b_r[...]
        B_r[...] = jnp.dot(hb, wb_r[...], preferred_element_type=jnp.float32)

    return pl.pallas_call(
        body,
        grid=(N // NB,),
        in_specs=[
            pl.BlockSpec((NB, H), lambda i: (i, 0)),
            pl.BlockSpec((H, H), lambda i: (0, 0)),
            pl.BlockSpec((H, H), lambda i: (0, 0)),
            pl.BlockSpec((1, H), lambda i: (0, 0)),
        ],
        out_specs=[
            pl.BlockSpec((NB, H), lambda i: (i, 0)),
            pl.BlockSpec((NB, H), lambda i: (i, 0)),
        ],
        out_shape=[jax.ShapeDtypeStruct((N, H), jnp.float32)] * 2,
    )(h, WiaT, WibT, be1)


def _sc_gather(A, B, pos, eic):
    QR = CH // 4

    @functools.partial(
        pl.kernel,
        out_type=[
            jax.ShapeDtypeStruct((E, H), jnp.float32),
            jax.ShapeDtypeStruct((E,), jnp.float32),
            jax.ShapeDtypeStruct((NCH, 3, CH), jnp.float32),
        ],
        mesh=_mesh,
        compiler_params=pltpu.CompilerParams(needs_layout_passes=False),
        scratch_types=[
            pltpu.VMEM((3 * N,), jnp.float32),
            pltpu.VMEM((2, CH), jnp.int32),
            pltpu.VMEM((2, CH), jnp.int32),
            pltpu.VMEM((CH, H), jnp.float32),
            pltpu.VMEM((CH, H), jnp.float32),
            pltpu.VMEM((CH, H), jnp.float32),
            pltpu.VMEM((CH, H), jnp.float32),
            pltpu.VMEM((CH,), jnp.float32),
            pltpu.VMEM((CH,), jnp.float32),
            pltpu.VMEM((3, CH), jnp.float32),
            pltpu.VMEM((3, CH), jnp.float32),
            pltpu.SemaphoreType.DMA,
            pltpu.SemaphoreType.DMA,
            pltpu.SemaphoreType.DMA,
            pltpu.SemaphoreType.DMA,
            pltpu.SemaphoreType.DMA,
            pltpu.SemaphoreType.DMA,
        ],
    )
    def k(A_h, B_h, pos_h, ei_h, G_h, r2_h, d3_h,
          posv, ib0, ib1, bA0, bA1, bB0, bB1, r2b0, r2b1, d3b0, d3b1,
          sA0, sA1, sB0, sB1, sW0, sW1):
        cid = lax.axis_index("c")
        sid = lax.axis_index("s")
        wid = sid * 2 + cid
        nk = 78 + jnp.where(wid < NCH - 78 * NW, 1, 0)
        pltpu.sync_copy(pos_h, posv)
        slots = ((ib0, bA0, bB0, r2b0, d3b0, sA0, sB0, sW0),
                 (ib1, bA1, bB1, r2b1, d3b1, sA1, sB1, sW1))

        def fetch(c, slot):
            ib, bA, bB, _, _, sA, sB, _ = slots[slot]
            g = wid + c * NW
            pltpu.sync_copy(ei_h.at[g], ib)
            pltpu.async_copy(A_h.at[ib.at[0]], bA, sA)
            pltpu.async_copy(B_h.at[ib.at[1]], bB, sB)

        def process(c, slot, prefetch):
            ib, bA, bB, r2b, d3b, sA, sB, sW = slots[slot]
            g = wid + c * NW
            base = g * CH
            for gi in range(CH // L):
                sl = pl.ds(gi * L, L)
                di = ib[0, sl] * 3
                si = ib[1, sl] * 3
                ddx = plsc.load_gather(posv, [di]) - plsc.load_gather(posv, [si])
                ddy = plsc.load_gather(posv, [di + 1]) - plsc.load_gather(posv, [si + 1])
                ddz = plsc.load_gather(posv, [di + 2]) - plsc.load_gather(posv, [si + 2])
                d3b[0, sl] = ddx
                d3b[1, sl] = ddy
                d3b[2, sl] = ddz
                r2b[sl] = ddx * ddx + ddy * ddy + ddz * ddz
            pltpu.make_async_copy(A_h.at[ib.at[0]], bA, sA).wait()
            pltpu.make_async_copy(B_h.at[ib.at[1]], bB, sB).wait()
            for q in range(4):
                def row(r, cc):
                    for qq in range(H // L):
                        s2 = pl.ds(qq * L, L)
                        bA[r, s2] = bA[r, s2] + bB[r, s2]
                    return cc

                lax.fori_loop(q * QR, (q + 1) * QR, row, 0)
                pltpu.async_copy(bA.at[pl.ds(q * QR, QR)],
                                 G_h.at[pl.ds(base + q * QR, QR)], sW)
            pltpu.sync_copy(r2b, r2_h.at[pl.ds(base, CH)])
            pltpu.sync_copy(d3b, d3_h.at[g])
            if prefetch:
                @pl.when(c + 2 < nk)
                def _():
                    for q in range(4):
                        pltpu.make_async_copy(
                            bA.at[pl.ds(q * QR, QR)],
                            G_h.at[pl.ds(base + q * QR, QR)], sW).wait()
                    fetch(c + 2, slot)

        fetch(0, 0)
        fetch(1, 1)

        def body(t, carry):
            process(2 * t, 0, True)
            process(2 * t + 1, 1, True)
            return carry

        lax.fori_loop(0, 39, body, 0)

        @pl.when(wid < NCH - 78 * NW)
        def _():
            process(78, 0, False)

        for q in range(4):
            pltpu.make_async_copy(bA0.at[pl.ds(q * QR, QR)],
                                  G_h.at[pl.ds(q * QR, QR)], sW0).wait()
            pltpu.make_async_copy(bA1.at[pl.ds(q * QR, QR)],
                                  G_h.at[pl.ds(q * QR, QR)], sW1).wait()

    return k(A, B, pos, eic)


def _tc_edge(G, ea, r2c, WeET, wr_row, We2T, be2, Wx1T, bx1, wx2_row, bx2):
    def body(G_r, ea_r, r2_r, weet_r, wr_r, we2t_r, be2_r, wx1t_r, bx1_r,
             wx2_r, bx2_r, m_r, gs_r):
        r2 = r2_r[...]
        pre1 = (G_r[...] + r2 * wr_r[...]
                + jnp.dot(ea_r[...], weet_r[...], preferred_element_type=jnp.float32))
        l1 = _silu(pre1).astype(jnp.bfloat16)
        mm = _silu(jnp.dot(l1, we2t_r[...], preferred_element_type=jnp.float32) + be2_r[...])
        m_r[...] = mm
        g1 = _silu(jnp.dot(mm.astype(jnp.bfloat16), wx1t_r[...],
                           preferred_element_type=jnp.float32) + bx1_r[...])
        gate = jnp.tanh(jnp.sum(g1 * wx2_r[...], axis=1, keepdims=True) + bx2_r[...])
        gs_r[...] = gate / (r2 + 1.0)

    return pl.pallas_call(
        body,
        grid=(E // BE,),
        in_specs=[
            pl.BlockSpec((BE, H), lambda i: (i, 0)),
            pl.BlockSpec((BE, 16), lambda i: (i, 0)),
            pl.BlockSpec((BE, 1), lambda i: (i, 0)),
            pl.BlockSpec((16, H), lambda i: (0, 0)),
            pl.BlockSpec((1, H), lambda i: (0, 0)),
            pl.BlockSpec((H, H), lambda i: (0, 0)),
            pl.BlockSpec((1, H), lambda i: (0, 0)),
            pl.BlockSpec((H, H), lambda i: (0, 0)),
            pl.BlockSpec((1, H), lambda i: (0, 0)),
            pl.BlockSpec((1, H), lambda i: (0, 0)),
            pl.BlockSpec((1, 1), lambda i: (0, 0)),
        ],
        out_specs=[
            pl.BlockSpec((BE, H), lambda i: (i, 0)),
            pl.BlockSpec((BE, 1), lambda i: (i, 0)),
        ],
        out_shape=[
            jax.ShapeDtypeStruct((E, H), jnp.float32),
            jax.ShapeDtypeStruct((E, 1), jnp.float32),
        ],
    )(G, ea, r2c, WeET, wr_row, We2T, be2, Wx1T, bx1, wx2_row, bx2)


def _sc_scatter(m, gs, d3, dst):
    @functools.partial(
        pl.kernel,
        out_type=[
            jax.ShapeDtypeStruct((2, NP, H), jnp.float32),
            jax.ShapeDtypeStruct((2, 3 * NP), jnp.float32),
        ],
        mesh=_mesh,
        compiler_params=pltpu.CompilerParams(needs_layout_passes=False),
        scratch_types=[
            pltpu.VMEM_SHARED((NP, H), jnp.float32),
            pltpu.VMEM_SHARED((3 * NP,), jnp.float32),
            pltpu.VMEM((CH,), jnp.int32),
            pltpu.VMEM((CH,), jnp.int32),
            pltpu.VMEM((CH, H), jnp.float32),
            pltpu.VMEM((CH, H), jnp.float32),
            pltpu.VMEM((CH,), jnp.float32),
            pltpu.VMEM((CH,), jnp.float32),
            pltpu.VMEM((3, CH), jnp.float32),
            pltpu.VMEM((3, CH), jnp.float32),
            pltpu.VMEM((3, CH), jnp.float32),
            pltpu.VMEM((3, CH), jnp.float32),
            pltpu.VMEM((3, CH), jnp.int32),
            pltpu.VMEM((3, CH), jnp.int32),
            pltpu.VMEM((3 * NP // 16,), jnp.float32),
            pltpu.SemaphoreType.DMA,
            pltpu.SemaphoreType.DMA,
            pltpu.SemaphoreType.DMA,
            pltpu.SemaphoreType.DMA,
        ],
    )
    def k(m_h, gs_h, d3_h, dst_h, agg_h, dp_h,
          agg_s, dp_s, dbuf0, dbuf1, mbuf0, mbuf1, gsb0, gsb1,
          d3c0, d3c1, cb0, cb1, ixb0, ixb1, dpb, sI0, sI1, sS0, sS1):
        cid = lax.axis_index("c")
        sid = lax.axis_index("s")
        wid = sid * 2 + cid
        nk = 78 + jnp.where(wid < NCH - 78 * NW, 1, 0)
        zv = jnp.zeros((L,), jnp.float32)
        DPW = 3 * NP // 16
        slots = ((dbuf0, mbuf0, gsb0, d3c0, cb0, ixb0, sI0, sS0),
                 (dbuf1, mbuf1, gsb1, d3c1, cb1, ixb1, sI1, sS1))

        def zrow(r, c):
            for q in range(H // L):
                mbuf0[r, pl.ds(q * L, L)] = zv
            return c

        lax.fori_loop(0, CH, zrow, 0)

        def zdp(r, c):
            dpb[pl.ds(r * L, L)] = zv
            return c

        lax.fori_loop(0, DPW // L, zdp, 0)
        for j in range(5):
            pltpu.sync_copy(mbuf0, agg_s.at[pl.ds(sid * 640 + j * RB, RB)])
        pltpu.sync_copy(dpb, dp_s.at[pl.ds(sid * DPW, DPW)])
        plsc.subcore_barrier()

        def fetch(c, slot):
            dbuf, mbuf, gsb, d3c, _, _, sI, _ = slots[slot]
            g = wid + c * NW
            base = g * CH
            pltpu.async_copy(dst_h.at[pl.ds(base, CH)], dbuf, sI)
            pltpu.async_copy(m_h.at[pl.ds(base, CH)], mbuf, sI)
            pltpu.async_copy(gs_h.at[pl.ds(base, CH)], gsb, sI)
            pltpu.async_copy(d3_h.at[g], d3c, sI)

        def process(c, slot, prefetch):
            dbuf, mbuf, gsb, d3c, cb, ixb, sI, sS = slots[slot]
            g = wid + c * NW
            base = g * CH
            pltpu.make_async_copy(dst_h.at[pl.ds(base, CH)], dbuf, sI).wait()
            pltpu.make_async_copy(m_h.at[pl.ds(base, CH)], mbuf, sI).wait()
            pltpu.make_async_copy(gs_h.at[pl.ds(base, CH)], gsb, sI).wait()
            pltpu.make_async_copy(d3_h.at[g], d3c, sI).wait()
            for gi in range(CH // L):
                sl = pl.ds(gi * L, L)
                gsv = gsb[sl]
                i3 = dbuf[sl] * 3
                ixb[0, sl] = i3
                ixb[1, sl] = i3 + 1
                ixb[2, sl] = i3 + 2
                cb[0, sl] = d3c[0, sl] * gsv
                cb[1, sl] = d3c[1, sl] * gsv
                cb[2, sl] = d3c[2, sl] * gsv
            pltpu.async_copy(mbuf, agg_s.at[dbuf], sS, add=True)
            pltpu.async_copy(cb.at[0], dp_s.at[ixb.at[0]], sS, add=True)
            pltpu.async_copy(cb.at[1], dp_s.at[ixb.at[1]], sS, add=True)
            pltpu.async_copy(cb.at[2], dp_s.at[ixb.at[2]], sS, add=True)
            if prefetch:
                @pl.when(c + 2 < nk)
                def _():
                    pltpu.make_async_copy(mbuf, agg_s.at[dbuf], sS).wait()
                    pltpu.make_async_copy(cb.at[0], dp_s.at[ixb.at[0]], sS).wait()
                    pltpu.make_async_copy(cb.at[1], dp_s.at[ixb.at[1]], sS).wait()
                    pltpu.make_async_copy(cb.at[2], dp_s.at[ixb.at[2]], sS).wait()
                    fetch(c + 2, slot)

        fetch(0, 0)
        fetch(1, 1)

        def body(t, carry):
            process(2 * t, 0, True)
            process(2 * t + 1, 1, True)
            return carry

        lax.fori_loop(0, 39, body, 0)

        @pl.when(wid < NCH - 78 * NW)
        def _():
            process(78, 0, False)

        for slot in range(2):
            dbuf, mbuf, _, _, cb, ixb, _, sS = slots[slot]
            pltpu.make_async_copy(mbuf, agg_s.at[dbuf], sS).wait()
            pltpu.make_async_copy(cb.at[0], dp_s.at[ixb.at[0]], sS).wait()
            pltpu.make_async_copy(cb.at[1], dp_s.at[ixb.at[1]], sS).wait()
            pltpu.make_async_copy(cb.at[2], dp_s.at[ixb.at[2]], sS).wait()
        plsc.subcore_barrier()
        for j in range(5):
            r0 = sid * 640 + j * RB
            pltpu.sync_copy(agg_s.at[pl.ds(r0, RB)], mbuf0)
            pltpu.sync_copy(mbuf0, agg_h.at[cid, pl.ds(r0, RB)])
        pltpu.sync_copy(dp_s.at[pl.ds(sid * DPW, DPW)], dpb)
        pltpu.sync_copy(dpb, dp_h.at[cid, pl.ds(sid * DPW, DPW)])

    return k(m, gs, d3, dst)


def _tc_node(h, agg2, dp2, pos, Wh1aT, Wh1bT, bh1, Wh2T, bh2, g, b):
    def body(h_r, agg_r, dp_r, pos_r, wa_r, wb_r, b1_r, w2_r, b2_r, g_r, be_r,
             ho_r, po_r):
        hb = h_r[...]
        a3 = agg_r[...]
        agg = a3[0] + a3[1]
        t = _silu(jnp.dot(hb, wa_r[...], preferred_element_type=jnp.float32)
                  + jnp.dot(agg, wb_r[...], preferred_element_type=jnp.float32)
                  + b1_r[...])
        dh = jnp.dot(t, w2_r[...], preferred_element_type=jnp.float32) + b2_r[...]
        x = hb + dh
        mu = jnp.mean(x, axis=1, keepdims=True)
        var = jnp.mean((x - mu) ** 2, axis=1, keepdims=True)
        ho_r[...] = (x - mu) * lax.rsqrt(var + 1e-5) * g_r[...] + be_r[...]
        d3 = dp_r[...]
        po_r[...] = pos_r[...] + d3[0] + d3[1]

    return pl.pallas_call(
        body,
        grid=(N // NB,),
        in_specs=[
            pl.BlockSpec((NB, H), lambda i: (i, 0)),
            pl.BlockSpec((2, NB, H), lambda i: (0, i, 0)),
            pl.BlockSpec((2, NB, 3), lambda i: (0, i, 0)),
            pl.BlockSpec((NB, 3), lambda i: (i, 0)),
            pl.BlockSpec((H, H), lambda i: (0, 0)),
            pl.BlockSpec((H, H), lambda i: (0, 0)),
            pl.BlockSpec((1, H), lambda i: (0, 0)),
            pl.BlockSpec((H, H), lambda i: (0, 0)),
            pl.BlockSpec((1, H), lambda i: (0, 0)),
            pl.BlockSpec((1, H), lambda i: (0, 0)),
            pl.BlockSpec((1, H), lambda i: (0, 0)),
        ],
        out_specs=[
            pl.BlockSpec((NB, H), lambda i: (i, 0)),
            pl.BlockSpec((NB, 3), lambda i: (i, 0)),
        ],
        out_shape=[
            jax.ShapeDtypeStruct((N, H), jnp.float32),
            jax.ShapeDtypeStruct((N, 3), jnp.float32),
        ],
    )(h, agg2, dp2, pos, Wh1aT, Wh1bT, bh1, Wh2T, bh2, g, b)


def kernel(h, pos, edge_index, edge_attr, We1_w, We1_b, We2_w, We2_b,
           Wh1_w, Wh1_b, Wh2_w, Wh2_b, Wx1_w, Wx1_b, Wx2_w, Wx2_b, ln_g, ln_b):
    src = edge_index[0].astype(jnp.int32)
    dst = edge_index[1].astype(jnp.int32)
    WiaT = We1_w[:, :H].T
    WibT = We1_w[:, H:2 * H].T
    wr_row = We1_w[:, 2 * H:2 * H + 1].T
    WeET = We1_w[:, 2 * H + 1:].T
    A, Bm = _tc_pre(h, WiaT, WibT, We1_b.reshape(1, H))
    eic = jnp.stack([dst, src]).reshape(2, NCH, CH).transpose(1, 0, 2)
    G, r2v, d3v = _sc_gather(A, Bm, pos.reshape(-1), eic)
    m, gs = _tc_edge(G, edge_attr.astype(jnp.bfloat16),
                     r2v.reshape(E, 1), WeET.astype(jnp.bfloat16), wr_row,
                     We2_w.T.astype(jnp.bfloat16), We2_b.reshape(1, H),
                     Wx1_w.T.astype(jnp.bfloat16), Wx1_b.reshape(1, H),
                     Wx2_w, Wx2_b.reshape(1, 1))
    agg2, dp2 = _sc_scatter(m, gs.reshape(E), d3v, dst)
    h_out, pos_out = _tc_node(h, agg2, dp2.reshape(2, NP, 3), pos, Wh1_w[:, :H].T, Wh1_w[:, H:].T,
                              Wh1_b.reshape(1, H), Wh2_w.T, Wh2_b.reshape(1, H),
                              ln_g.reshape(1, H), ln_b.reshape(1, H))
    return (h_out, pos_out)


# speedup vs baseline: 5.8883x; 1.0348x over previous
import functools

import jax
import jax.numpy as jnp
from jax import lax
from jax.experimental import pallas as pl
from jax.experimental.pallas import tpu as pltpu
from jax.experimental.pallas import tpu_sc as plsc

H = 128
N = 10000
E = 320000
L = 16
CH = 128
NW = 32
NCH = E // CH
NP = 10240
RB = 128
NB = 1000
BE = 4000

_mesh = plsc.VectorSubcoreMesh(core_axis_name="c", subcore_axis_name="s")


def _silu(x):
    return x * jax.nn.sigmoid(x)


def _tc_pre(h, WiaT, WibT, be1):
    def body(h_r, wa_r, wb_r, b_r, A_r, B_r):
        hb = h_r[...]
        A_r[...] = jnp.dot(hb, wa_r[...], preferred_element_type=jnp.float32) + b_r[...]
        B_r[...] = jnp.dot(hb, wb_r[...], preferred_element_type=jnp.float32)

    return pl.pallas_call(
        body,
        grid=(N // NB,),
        in_specs=[
            pl.BlockSpec((NB, H), lambda i: (i, 0)),
            pl.BlockSpec((H, H), lambda i: (0, 0)),
            pl.BlockSpec((H, H), lambda i: (0, 0)),
            pl.BlockSpec((1, H), lambda i: (0, 0)),
        ],
        out_specs=[
            pl.BlockSpec((NB, H), lambda i: (i, 0)),
            pl.BlockSpec((NB, H), lambda i: (i, 0)),
        ],
        out_shape=[jax.ShapeDtypeStruct((N, H), jnp.float32)] * 2,
    )(h, WiaT, WibT, be1)


def _sc_gather(A, B, pos, eic):
    QR = CH // 4

    @functools.partial(
        pl.kernel,
        out_type=[
            jax.ShapeDtypeStruct((E, H), jnp.float32),
            jax.ShapeDtypeStruct((E,), jnp.float32),
            jax.ShapeDtypeStruct((NCH, 3, CH), jnp.float32),
        ],
        mesh=_mesh,
        compiler_params=pltpu.CompilerParams(needs_layout_passes=False),
        scratch_types=[
            pltpu.VMEM((3 * N,), jnp.float32),
            pltpu.VMEM((2, CH), jnp.int32),
            pltpu.VMEM((2, CH), jnp.int32),
            pltpu.VMEM((CH, H), jnp.float32),
            pltpu.VMEM((CH, H), jnp.float32),
            pltpu.VMEM((CH, H), jnp.float32),
            pltpu.VMEM((CH, H), jnp.float32),
            pltpu.VMEM((CH,), jnp.float32),
            pltpu.VMEM((CH,), jnp.float32),
            pltpu.VMEM((3, CH), jnp.float32),
            pltpu.VMEM((3, CH), jnp.float32),
            pltpu.SemaphoreType.DMA,
            pltpu.SemaphoreType.DMA,
            pltpu.SemaphoreType.DMA,
            pltpu.SemaphoreType.DMA,
            pltpu.SemaphoreType.DMA,
            pltpu.SemaphoreType.DMA,
        ],
    )
    def k(A_h, B_h, pos_h, ei_h, G_h, r2_h, d3_h,
          posv, ib0, ib1, bA0, bA1, bB0, bB1, r2b0, r2b1, d3b0, d3b1,
          sA0, sA1, sB0, sB1, sW0, sW1):
        cid = lax.axis_index("c")
        sid = lax.axis_index("s")
        wid = sid * 2 + cid
        nk = 78 + jnp.where(wid < NCH - 78 * NW, 1, 0)
        pltpu.sync_copy(pos_h, posv)
        slots = ((ib0, bA0, bB0, r2b0, d3b0, sA0, sB0, sW0),
                 (ib1, bA1, bB1, r2b1, d3b1, sA1, sB1, sW1))

        def fetch(c, slot):
            ib, bA, bB, _, _, sA, sB, _ = slots[slot]
            g = wid + c * NW
            pltpu.sync_copy(ei_h.at[g], ib)
            pltpu.async_copy(A_h.at[ib.at[0]], bA, sA)
            pltpu.async_copy(B_h.at[ib.at[1]], bB, sB)

        def process(c, slot, prefetch):
            ib, bA, bB, r2b, d3b, sA, sB, sW = slots[slot]
            g = wid + c * NW
            base = g * CH
            for gi in range(CH // L):
                sl = pl.ds(gi * L, L)
                di = ib[0, sl] * 3
                si = ib[1, sl] * 3
                ddx = plsc.load_gather(posv, [di]) - plsc.load_gather(posv, [si])
                ddy = plsc.load_gather(posv, [di + 1]) - plsc.load_gather(posv, [si + 1])
                ddz = plsc.load_gather(posv, [di + 2]) - plsc.load_gather(posv, [si + 2])
                d3b[0, sl] = ddx
                d3b[1, sl] = ddy
                d3b[2, sl] = ddz
                r2b[sl] = ddx * ddx + ddy * ddy + ddz * ddz
            pltpu.make_async_copy(A_h.at[ib.at[0]], bA, sA).wait()
            pltpu.make_async_copy(B_h.at[ib.at[1]], bB, sB).wait()
            for q in range(4):
                def row(r, cc):
                    for qq in range(H // L):
                        s2 = pl.ds(qq * L, L)
                        bA[r, s2] = bA[r, s2] + bB[r, s2]
                    return cc

                lax.fori_loop(q * QR, (q + 1) * QR, row, 0)
                pltpu.async_copy(bA.at[pl.ds(q * QR, QR)],
                                 G_h.at[pl.ds(base + q * QR, QR)], sW)
            pltpu.sync_copy(r2b, r2_h.at[pl.ds(base, CH)])
            pltpu.sync_copy(d3b, d3_h.at[g])
            if prefetch:
                @pl.when(c + 2 < nk)
                def _():
                    for q in range(4):
                        pltpu.make_async_copy(
                            bA.at[pl.ds(q * QR, QR)],
                            G_h.at[pl.ds(base + q * QR, QR)], sW).wait()
                    fetch(c + 2, slot)

        fetch(0, 0)
        fetch(1, 1)

        def body(t, carry):
            process(2 * t, 0, True)
            process(2 * t + 1, 1, True)
            return carry

        lax.fori_loop(0, 39, body, 0)

        @pl.when(wid < NCH - 78 * NW)
        def _():
            process(78, 0, False)

        for q in range(4):
            pltpu.make_async_copy(bA0.at[pl.ds(q * QR, QR)],
                                  G_h.at[pl.ds(q * QR, QR)], sW0).wait()
            pltpu.make_async_copy(bA1.at[pl.ds(q * QR, QR)],
                                  G_h.at[pl.ds(q * QR, QR)], sW1).wait()

    return k(A, B, pos, eic)


def _tc_edge(G, ea, r2c, WeET, wr_row, We2T, be2, Wx1T, bx1, wx2_row, bx2):
    def body(G_r, ea_r, r2_r, weet_r, wr_r, we2t_r, be2_r, wx1t_r, bx1_r,
             wx2_r, bx2_r, m_r, gs_r):
        r2 = r2_r[...]
        pre1 = (G_r[...] + r2 * wr_r[...]
                + jnp.dot(ea_r[...], weet_r[...], preferred_element_type=jnp.float32))
        l1 = _silu(pre1).astype(jnp.bfloat16)
        mm = _silu(jnp.dot(l1, we2t_r[...], preferred_element_type=jnp.float32) + be2_r[...])
        m_r[...] = mm
        g1 = _silu(jnp.dot(mm.astype(jnp.bfloat16), wx1t_r[...],
                           preferred_element_type=jnp.float32) + bx1_r[...])
        gate = jnp.tanh(jnp.sum(g1 * wx2_r[...], axis=1, keepdims=True) + bx2_r[...])
        gs_r[...] = gate / (r2 + 1.0)

    return pl.pallas_call(
        body,
        grid=(E // BE,),
        in_specs=[
            pl.BlockSpec((BE, H), lambda i: (i, 0)),
            pl.BlockSpec((BE, 16), lambda i: (i, 0)),
            pl.BlockSpec((BE, 1), lambda i: (i, 0)),
            pl.BlockSpec((16, H), lambda i: (0, 0)),
            pl.BlockSpec((1, H), lambda i: (0, 0)),
            pl.BlockSpec((H, H), lambda i: (0, 0)),
            pl.BlockSpec((1, H), lambda i: (0, 0)),
            pl.BlockSpec((H, H), lambda i: (0, 0)),
            pl.BlockSpec((1, H), lambda i: (0, 0)),
            pl.BlockSpec((1, H), lambda i: (0, 0)),
            pl.BlockSpec((1, 1), lambda i: (0, 0)),
        ],
        out_specs=[
            pl.BlockSpec((BE, H), lambda i: (i, 0)),
            pl.BlockSpec((BE, 1), lambda i: (i, 0)),
        ],
        out_shape=[
            jax.ShapeDtypeStruct((E, H), jnp.float32),
            jax.ShapeDtypeStruct((E, 1), jnp.float32),
        ],
    )(G, ea, r2c, WeET, wr_row, We2T, be2, Wx1T, bx1, wx2_row, bx2)


def _sc_scatter(m, gs, d3, dst):
    @functools.partial(
        pl.kernel,
        out_type=[
            jax.ShapeDtypeStruct((2, NP, H), jnp.float32),
            jax.ShapeDtypeStruct((2, 3 * NP), jnp.float32),
        ],
        mesh=_mesh,
        compiler_params=pltpu.CompilerParams(needs_layout_passes=False),
        scratch_types=[
            pltpu.VMEM_SHARED((NP, H), jnp.float32),
            pltpu.VMEM_SHARED((3 * NP,), jnp.float32),
            pltpu.VMEM((CH,), jnp.int32),
            pltpu.VMEM((CH,), jnp.int32),
            pltpu.VMEM((CH, H), jnp.float32),
            pltpu.VMEM((CH, H), jnp.float32),
            pltpu.VMEM((CH,), jnp.float32),
            pltpu.VMEM((CH,), jnp.float32),
            pltpu.VMEM((3, CH), jnp.float32),
            pltpu.VMEM((3, CH), jnp.float32),
            pltpu.VMEM((3, CH), jnp.float32),
            pltpu.VMEM((3, CH), jnp.float32),
            pltpu.VMEM((3, CH), jnp.int32),
            pltpu.VMEM((3, CH), jnp.int32),
            pltpu.VMEM((3 * NP // 16,), jnp.float32),
            pltpu.SemaphoreType.DMA,
            pltpu.SemaphoreType.DMA,
            pltpu.SemaphoreType.DMA,
            pltpu.SemaphoreType.DMA,
        ],
    )
    def k(m_h, gs_h, d3_h, dst_h, agg_h, dp_h,
          agg_s, dp_s, dbuf0, dbuf1, mbuf0, mbuf1, gsb0, gsb1,
          d3c0, d3c1, cb0, cb1, ixb0, ixb1, dpb, sI0, sI1, sS0, sS1):
        cid = lax.axis_index("c")
        sid = lax.axis_index("s")
        wid = sid * 2 + cid
        nk = 78 + jnp.where(wid < NCH - 78 * NW, 1, 0)
        zv = jnp.zeros((L,), jnp.float32)
        DPW = 3 * NP // 16
        slots = ((dbuf0, mbuf0, gsb0, d3c0, cb0, ixb0, sI0, sS0),
                 (dbuf1, mbuf1, gsb1, d3c1, cb1, ixb1, sI1, sS1))

        def zrow(r, c):
            for q in range(H // L):
                mbuf0[r, pl.ds(q * L, L)] = zv
            return c

        lax.fori_loop(0, CH, zrow, 0)

        def zdp(r, c):
            dpb[pl.ds(r * L, L)] = zv
            return c

        lax.fori_loop(0, DPW // L, zdp, 0)
        for j in range(5):
            pltpu.sync_copy(mbuf0, agg_s.at[pl.ds(sid * 640 + j * RB, RB)])
        pltpu.sync_copy(dpb, dp_s.at[pl.ds(sid * DPW, DPW)])
        plsc.subcore_barrier()

        def fetch(c, slot):
            dbuf, mbuf, gsb, d3c, _, _, sI, _ = slots[slot]
            g = wid + c * NW
            base = g * CH
            pltpu.async_copy(dst_h.at[pl.ds(base, CH)], dbuf, sI)
            pltpu.async_copy(m_h.at[pl.ds(base, CH)], mbuf, sI)
            pltpu.async_copy(gs_h.at[pl.ds(base, CH)], gsb, sI)
            pltpu.async_copy(d3_h.at[g], d3c, sI)

        def process(c, slot, prefetch):
            dbuf, mbuf, gsb, d3c, cb, ixb, sI, sS = slots[slot]
            g = wid + c * NW
            base = g * CH
            pltpu.make_async_copy(dst_h.at[pl.ds(base, CH)], dbuf, sI).wait()
            pltpu.make_async_copy(m_h.at[pl.ds(base, CH)], mbuf, sI).wait()
            pltpu.make_async_copy(gs_h.at[pl.ds(base, CH)], gsb, sI).wait()
            pltpu.make_async_copy(d3_h.at[g], d3c, sI).wait()
            for gi in range(CH // L):
                sl = pl.ds(gi * L, L)
                gsv = gsb[sl]
                i3 = dbuf[sl] * 3
                ixb[0, sl] = i3
                ixb[1, sl] = i3 + 1
                ixb[2, sl] = i3 + 2
                cb[0, sl] = d3c[0, sl] * gsv
                cb[1, sl] = d3c[1, sl] * gsv
                cb[2, sl] = d3c[2, sl] * gsv
            pltpu.async_copy(mbuf, agg_s.at[dbuf], sS, add=True)
            pltpu.async_copy(cb.at[0], dp_s.at[ixb.at[0]], sS, add=True)
            pltpu.async_copy(cb.at[1], dp_s.at[ixb.at[1]], sS, add=True)
            pltpu.async_copy(cb.at[2], dp_s.at[ixb.at[2]], sS, add=True)
            if prefetch:
                @pl.when(c + 2 < nk)
                def _():
                    pltpu.make_async_copy(mbuf, agg_s.at[dbuf], sS).wait()
                    pltpu.make_async_copy(cb.at[0], dp_s.at[ixb.at[0]], sS).wait()
                    pltpu.make_async_copy(cb.at[1], dp_s.at[ixb.at[1]], sS).wait()
                    pltpu.make_async_copy(cb.at[2], dp_s.at[ixb.at[2]], sS).wait()
                    fetch(c + 2, slot)

        fetch(0, 0)
        fetch(1, 1)

        def body(t, carry):
            process(2 * t, 0, True)
            process(2 * t + 1, 1, True)
            return carry

        lax.fori_loop(0, 39, body, 0)

        @pl.when(wid < NCH - 78 * NW)
        def _():
            process(78, 0, False)

        for slot in range(2):
            dbuf, mbuf, _, _, cb, ixb, _, sS = slots[slot]
            pltpu.make_async_copy(mbuf, agg_s.at[dbuf], sS).wait()
            pltpu.make_async_copy(cb.at[0], dp_s.at[ixb.at[0]], sS).wait()
            pltpu.make_async_copy(cb.at[1], dp_s.at[ixb.at[1]], sS).wait()
            pltpu.make_async_copy(cb.at[2], dp_s.at[ixb.at[2]], sS).wait()
        plsc.subcore_barrier()
        for j in range(5):
            r0 = sid * 640 + j * RB
            pltpu.sync_copy(agg_s.at[pl.ds(r0, RB)], mbuf0)
            pltpu.sync_copy(mbuf0, agg_h.at[cid, pl.ds(r0, RB)])
        pltpu.sync_copy(dp_s.at[pl.ds(sid * DPW, DPW)], dpb)
        pltpu.sync_copy(dpb, dp_h.at[cid, pl.ds(sid * DPW, DPW)])

    return k(m, gs, d3, dst)


def _tc_node(h, agg2, dp2, pos, Wh1aT, Wh1bT, bh1, Wh2T, bh2, g, b):
    def body(h_r, agg_r, dp_r, pos_r, wa_r, wb_r, b1_r, w2_r, b2_r, g_r, be_r,
             ho_r, po_r):
        hb = h_r[...]
        a3 = agg_r[...]
        agg = a3[0] + a3[1]
        t = _silu(jnp.dot(hb, wa_r[...], preferred_element_type=jnp.float32)
                  + jnp.dot(agg, wb_r[...], preferred_element_type=jnp.float32)
                  + b1_r[...])
        dh = jnp.dot(t, w2_r[...], preferred_element_type=jnp.float32) + b2_r[...]
        x = hb + dh
        mu = jnp.mean(x, axis=1, keepdims=True)
        var = jnp.mean((x - mu) ** 2, axis=1, keepdims=True)
        ho_r[...] = (x - mu) * lax.rsqrt(var + 1e-5) * g_r[...] + be_r[...]
        d3 = dp_r[...]
        po_r[...] = pos_r[...] + d3[0] + d3[1]

    return pl.pallas_call(
        body,
        grid=(N // NB,),
        in_specs=[
            pl.BlockSpec((NB, H), lambda i: (i, 0)),
            pl.BlockSpec((2, NB, H), lambda i: (0, i, 0)),
            pl.BlockSpec((2, NB, 3), lambda i: (0, i, 0)),
            pl.BlockSpec((NB, 3), lambda i: (i, 0)),
            pl.BlockSpec((H, H), lambda i: (0, 0)),
            pl.BlockSpec((H, H), lambda i: (0, 0)),
            pl.BlockSpec((1, H), lambda i: (0, 0)),
            pl.BlockSpec((H, H), lambda i: (0, 0)),
            pl.BlockSpec((1, H), lambda i: (0, 0)),
            pl.BlockSpec((1, H), lambda i: (0, 0)),
            pl.BlockSpec((1, H), lambda i: (0, 0)),
        ],
        out_specs=[
            pl.BlockSpec((NB, H), lambda i: (i, 0)),
            pl.BlockSpec((NB, 3), lambda i: (i, 0)),
        ],
        out_shape=[
            jax.ShapeDtypeStruct((N, H), jnp.float32),
            jax.ShapeDtypeStruct((N, 3), jnp.float32),
        ],
    )(h, agg2, dp2, pos, Wh1aT, Wh1bT, bh1, Wh2T, bh2, g, b)


def kernel(h, pos, edge_index, edge_attr, We1_w, We1_b, We2_w, We2_b,
           Wh1_w, Wh1_b, Wh2_w, Wh2_b, Wx1_w, Wx1_b, Wx2_w, Wx2_b, ln_g, ln_b):
    src = edge_index[0].astype(jnp.int32)
    dst = edge_index[1].astype(jnp.int32)
    WiaT = We1_w[:, :H].T
    WibT = We1_w[:, H:2 * H].T
    wr_row = We1_w[:, 2 * H:2 * H + 1].T
    WeET = We1_w[:, 2 * H + 1:].T
    A, Bm = _tc_pre(h, WiaT, WibT, We1_b.reshape(1, H))
    eic = jnp.stack([dst, src]).reshape(2, NCH, CH).transpose(1, 0, 2)
    G, r2v, d3v = _sc_gather(A, Bm, pos.reshape(-1), eic)
    m, gs = _tc_edge(G, edge_attr.astype(jnp.bfloat16),
                     r2v.reshape(E, 1), WeET.astype(jnp.bfloat16), wr_row,
                     We2_w.T.astype(jnp.bfloat16), We2_b.reshape(1, H),
                     Wx1_w.T.astype(jnp.bfloat16), Wx1_b.reshape(1, H),
                     Wx2_w, Wx2_b.reshape(1, 1))
    agg2, dp2 = _sc_scatter(m, gs.reshape(E), d3v, dst)
    h_out, pos_out = _tc_node(h, agg2, dp2.reshape(2, NP, 3), pos, Wh1_w[:, :H].T, Wh1_w[:, H:].T,
                              Wh1_b.reshape(1, H), Wh2_w.T, Wh2_b.reshape(1, H),
                              ln_g.reshape(1, H), ln_b.reshape(1, H))
    return (h_out, pos_out)
